# baseline (speedup 1.0000x reference)
import functools

import jax
import jax.numpy as jnp
from jax import lax
from jax.experimental import pallas as pl
from jax.experimental.pallas import tpu as pltpu
from jax.experimental.pallas import tpu_sc as plsc

_N = 10000
_E = 320000
_H = 128
_C = 40
_NC = 2
_NS = 16
_NW = _NC * _NS
_CHUNK = 128
_CPT = 80
_NCH = _NW * _CPT
_EPT = _E // _NW
_NP = 10240
_RPT = _NP // _NS
_ZR = 128
_NBUF = 2
_ROWB = 2000

_sc_mesh = plsc.VectorSubcoreMesh(core_axis_name="c", subcore_axis_name="s")


@functools.partial(
    pl.kernel,
    out_type=jax.ShapeDtypeStruct((_NC, _NP, _H), jnp.float32),
    mesh=_sc_mesh,
    scratch_types=[
        pltpu.VMEM((_CPT, _CHUNK), jnp.int32),
        pltpu.VMEM((_CHUNK, _H), jnp.float32),
        pltpu.VMEM((_ZR, _H), jnp.float32),
        pltpu.VMEM_SHARED((_NP, _H), jnp.float32),
        pltpu.SemaphoreType.DMA,
        pltpu.SemaphoreType.DMA,
        pltpu.SemaphoreType.DMA,
        pltpu.SemaphoreType.DMA,
    ],
)
def _deg_call(dst2_hbm, out_hbm, sdst, ones_v, zbuf, acc_sh, t0, t1, t2, t3):
    c = lax.axis_index("c")
    s = lax.axis_index("s")
    wid = c * _NS + s
    ssem = (t0, t1, t2, t3)

    pltpu.sync_copy(dst2_hbm.at[pl.ds(wid * _CPT, _CPT), :], sdst)

    @pl.loop(0, _ZR)
    def _(i):
        for j in range(_H // 16):
            zbuf[i, pl.ds(j * 16, 16)] = jnp.zeros((16,), jnp.float32)

    @pl.loop(0, _CHUNK)
    def _(i):
        for j in range(_H // 16):
            ones_v[i, pl.ds(j * 16, 16)] = jnp.full((16,), 1.0, jnp.float32)

    for k in range(_RPT // _ZR):
        pltpu.sync_copy(zbuf, acc_sh.at[pl.ds(s * _RPT + k * _ZR, _ZR), :])
    plsc.subcore_barrier()

    for b in range(_NBUF):
        pltpu.async_copy(ones_v, acc_sh.at[sdst.at[b]], ssem[b], add=True)

    @pl.loop(0, _CPT - _NBUF, step=_NBUF)
    def _(i):
        for b in range(_NBUF):
            j = i + b
            pltpu.make_async_copy(ones_v, acc_sh.at[sdst.at[j]], ssem[b]).wait()
            pltpu.async_copy(ones_v, acc_sh.at[sdst.at[j + _NBUF]], ssem[b],
                             add=True)

    for b in range(_NBUF):
        j = _CPT - _NBUF + b
        pltpu.make_async_copy(ones_v, acc_sh.at[sdst.at[j]], ssem[b]).wait()

    plsc.subcore_barrier()
    pltpu.sync_copy(acc_sh.at[pl.ds(s * _RPT, _RPT), :],
                    out_hbm.at[c, pl.ds(s * _RPT, _RPT), :])


@functools.partial(
    pl.kernel,
    out_type=jax.ShapeDtypeStruct((_NC, _NP, _H), jnp.float32),
    mesh=_sc_mesh,
    scratch_types=[
        pltpu.VMEM((_CPT, _CHUNK), jnp.int32),
        pltpu.VMEM((_CPT, _CHUNK), jnp.int32),
        pltpu.VMEM((_CHUNK, _H), jnp.float32),
        pltpu.VMEM_SHARED((_NP, _H), jnp.float32),
        pltpu.SemaphoreType.DMA,
        pltpu.SemaphoreType.DMA,
    ],
)
def _agg_call(hp_hbm, src2_hbm, dst2_hbm, out_hbm, ssrc, sdst, rows,
              acc_sh, gsem, tsem):
    c = lax.axis_index("c")
    s = lax.axis_index("s")
    wid = c * _NS + s

    pltpu.sync_copy(src2_hbm.at[pl.ds(wid * _CPT, _CPT), :], ssrc)
    pltpu.sync_copy(dst2_hbm.at[pl.ds(wid * _CPT, _CPT), :], sdst)

    @pl.loop(0, _CHUNK)
    def _(i):
        for j in range(_H // 16):
            rows[i, pl.ds(j * 16, 16)] = jnp.zeros((16,), jnp.float32)

    for k in range(_RPT // _ZR):
        pltpu.sync_copy(rows, acc_sh.at[pl.ds(s * _RPT + k * _ZR, _ZR), :])
    plsc.subcore_barrier()

    @pl.loop(0, _CPT)
    def _(j):
        pltpu.async_copy(hp_hbm.at[ssrc.at[j]], rows, gsem).wait()
        pltpu.async_copy(rows, acc_sh.at[sdst.at[j]], tsem, add=True).wait()

    plsc.subcore_barrier()
    pltpu.sync_copy(acc_sh.at[pl.ds(s * _RPT, _RPT), :],
                    out_hbm.at[c, pl.ds(s * _RPT, _RPT), :])


_PREC = lax.Precision.HIGHEST


def _mm1_body(p0_ref, p1_ref, x_ref, w_ref, oh_ref, od_ref):
    deg = 1.0 + p0_ref[...] + p1_ref[...]
    dinv = lax.rsqrt(deg)
    g = jnp.dot(x_ref[...], w_ref[...],
                preferred_element_type=jnp.float32, precision=_PREC)
    oh_ref[...] = g * dinv
    od_ref[...] = dinv


@jax.jit
def _mm1_call(p0, p1, x, w):
    grid = (_N // _ROWB,)
    return pl.pallas_call(
        _mm1_body,
        grid=grid,
        in_specs=[
            pl.BlockSpec((_ROWB, 1), lambda i: (i, 0)),
            pl.BlockSpec((_ROWB, 1), lambda i: (i, 0)),
            pl.BlockSpec((_ROWB, _H), lambda i: (i, 0)),
            pl.BlockSpec((_H, _H), lambda i: (0, 0)),
        ],
        out_specs=[
            pl.BlockSpec((_ROWB, _H), lambda i: (i, 0)),
            pl.BlockSpec((_ROWB, 1), lambda i: (i, 0)),
        ],
        out_shape=[
            jax.ShapeDtypeStruct((_N, _H), jnp.float32),
            jax.ShapeDtypeStruct((_N, 1), jnp.float32),
        ],
    )(p0, p1, x, w)


def _layer_body(s0_ref, s1_ref, hp_ref, d_ref, b_ref, w_ref, o_ref):
    y = d_ref[...] * (s0_ref[...] + s1_ref[...] + hp_ref[...]) + b_ref[...]
    y = jnp.maximum(y, 0.0)
    o_ref[...] = jnp.dot(y, w_ref[...],
                         preferred_element_type=jnp.float32,
                         precision=_PREC) * d_ref[...]


@jax.jit
def _layer_call(s0, s1, hp, dinv, b, w):
    grid = (_N // _ROWB,)
    return pl.pallas_call(
        _layer_body,
        grid=grid,
        in_specs=[
            pl.BlockSpec((_ROWB, _H), lambda i: (i, 0)),
            pl.BlockSpec((_ROWB, _H), lambda i: (i, 0)),
            pl.BlockSpec((_ROWB, _H), lambda i: (i, 0)),
            pl.BlockSpec((_ROWB, 1), lambda i: (i, 0)),
            pl.BlockSpec((1, _H), lambda i: (0, 0)),
            pl.BlockSpec((_H, _H), lambda i: (0, 0)),
        ],
        out_specs=pl.BlockSpec((_ROWB, _H), lambda i: (i, 0)),
        out_shape=jax.ShapeDtypeStruct((_N, _H), jnp.float32),
    )(s0, s1, hp, dinv, b, w)


def _final_body(s0_ref, s1_ref, hp_ref, d_ref, b_ref, wl_ref, bl_ref, o_ref):
    y = d_ref[...] * (s0_ref[...] + s1_ref[...] + hp_ref[...]) + b_ref[...]
    y = jnp.maximum(y, 0.0)
    o_ref[...] = jnp.dot(y, wl_ref[...],
                         preferred_element_type=jnp.float32,
                         precision=_PREC) + bl_ref[...]


@jax.jit
def _final_call(s0, s1, hp, dinv, b, wl, bl):
    grid = (_N // _ROWB,)
    return pl.pallas_call(
        _final_body,
        grid=grid,
        in_specs=[
            pl.BlockSpec((_ROWB, _H), lambda i: (i, 0)),
            pl.BlockSpec((_ROWB, _H), lambda i: (i, 0)),
            pl.BlockSpec((_ROWB, _H), lambda i: (i, 0)),
            pl.BlockSpec((_ROWB, 1), lambda i: (i, 0)),
            pl.BlockSpec((1, _H), lambda i: (0, 0)),
            pl.BlockSpec((_H, _C), lambda i: (0, 0)),
            pl.BlockSpec((1, _C), lambda i: (0, 0)),
        ],
        out_specs=pl.BlockSpec((_ROWB, _C), lambda i: (i, 0)),
        out_shape=jax.ShapeDtypeStruct((_N, _C), jnp.float32),
    )(s0, s1, hp, dinv, b, wl, bl)


@jax.jit
def kernel(x, edge_index, W1, b1, W2, b2, W3, b3, Wl, bl):
    src = edge_index[0]
    dst = edge_index[1]
    pad = _CPT * _CHUNK - _EPT
    src2 = jnp.pad(src.reshape(_NW, _EPT), ((0, 0), (0, pad))
                   ).reshape(_NCH, _CHUNK)
    dst2 = jnp.pad(dst.reshape(_NW, _EPT), ((0, 0), (0, pad)),
                   constant_values=_NP - 1).reshape(_NCH, _CHUNK)
    degp = _deg_call(dst2)
    p0 = degp[0, :_N, :1]
    p1 = degp[1, :_N, :1]
    h1, dinv = _mm1_call(p0, p1, x, W1)
    s = _agg_call(h1, src2, dst2)
    h2 = _layer_call(s[0, :_N], s[1, :_N], h1, dinv, b1.reshape(1, _H), W2)
    s = _agg_call(h2, src2, dst2)
    h3 = _layer_call(s[0, :_N], s[1, :_N], h2, dinv, b2.reshape(1, _H), W3)
    s = _agg_call(h3, src2, dst2)
    out = _final_call(s[0, :_N], s[1, :_N], h3, dinv, b3.reshape(1, _H), Wl,
                      bl.reshape(1, _C))
    return out


# speedup vs baseline: 1.0276x; 1.0276x over previous
import functools

import jax
import jax.numpy as jnp
from jax import lax
from jax.experimental import pallas as pl
from jax.experimental.pallas import tpu as pltpu
from jax.experimental.pallas import tpu_sc as plsc

_N = 10000
_E = 320000
_H = 128
_C = 40
_NC = 2
_NS = 16
_NW = _NC * _NS
_CHUNK = 128
_CPT = 80
_NCH = _NW * _CPT
_EPT = _E // _NW
_NP = 10240
_RPT = _NP // _NS
_ZR = 128
_NBUF = 2
_ROWB = 2000

_sc_mesh = plsc.VectorSubcoreMesh(core_axis_name="c", subcore_axis_name="s")


@functools.partial(
    pl.kernel,
    out_type=jax.ShapeDtypeStruct((_NC, _NP, _H), jnp.float32),
    mesh=_sc_mesh,
    scratch_types=[
        pltpu.VMEM((_CPT, _CHUNK), jnp.int32),
        pltpu.VMEM((_CHUNK, _H), jnp.float32),
        pltpu.VMEM((_ZR, _H), jnp.float32),
        pltpu.VMEM_SHARED((_NP, _H), jnp.float32),
        pltpu.SemaphoreType.DMA,
        pltpu.SemaphoreType.DMA,
        pltpu.SemaphoreType.DMA,
        pltpu.SemaphoreType.DMA,
    ],
)
def _deg_call(dst2_hbm, out_hbm, sdst, ones_v, zbuf, acc_sh, t0, t1, t2, t3):
    c = lax.axis_index("c")
    s = lax.axis_index("s")
    wid = c * _NS + s
    ssem = (t0, t1, t2, t3)

    pltpu.sync_copy(dst2_hbm.at[pl.ds(wid * _CPT, _CPT), :], sdst)

    @pl.loop(0, _ZR)
    def _(i):
        for j in range(_H // 16):
            zbuf[i, pl.ds(j * 16, 16)] = jnp.zeros((16,), jnp.float32)

    @pl.loop(0, _CHUNK)
    def _(i):
        for j in range(_H // 16):
            ones_v[i, pl.ds(j * 16, 16)] = jnp.full((16,), 1.0, jnp.float32)

    for k in range(_RPT // _ZR):
        pltpu.sync_copy(zbuf, acc_sh.at[pl.ds(s * _RPT + k * _ZR, _ZR), :])
    plsc.subcore_barrier()

    for b in range(_NBUF):
        pltpu.async_copy(ones_v, acc_sh.at[sdst.at[b]], ssem[b], add=True)

    @pl.loop(0, _CPT - _NBUF, step=_NBUF)
    def _(i):
        for b in range(_NBUF):
            j = i + b
            pltpu.make_async_copy(ones_v, acc_sh.at[sdst.at[j]], ssem[b]).wait()
            pltpu.async_copy(ones_v, acc_sh.at[sdst.at[j + _NBUF]], ssem[b],
                             add=True)

    for b in range(_NBUF):
        j = _CPT - _NBUF + b
        pltpu.make_async_copy(ones_v, acc_sh.at[sdst.at[j]], ssem[b]).wait()

    plsc.subcore_barrier()
    pltpu.sync_copy(acc_sh.at[pl.ds(s * _RPT, _RPT), :],
                    out_hbm.at[c, pl.ds(s * _RPT, _RPT), :])


@functools.partial(
    pl.kernel,
    out_type=jax.ShapeDtypeStruct((_NC, _NP, _H), jnp.float32),
    mesh=_sc_mesh,
    scratch_types=[
        pltpu.VMEM((_CPT, _CHUNK), jnp.int32),
        pltpu.VMEM((_CHUNK,), jnp.int32),
        pltpu.VMEM((_CHUNK,), jnp.int32),
        pltpu.VMEM((_CHUNK, _H), jnp.float32),
        pltpu.VMEM((_CHUNK, _H), jnp.float32),
        pltpu.VMEM_SHARED((_NP, _H), jnp.float32),
        pltpu.SemaphoreType.DMA,
        pltpu.SemaphoreType.DMA,
        pltpu.SemaphoreType.DMA,
        pltpu.SemaphoreType.DMA,
    ],
)
def _agg_call(hp_hbm, srcf_hbm, dst2_hbm, out_hbm, sdst, i0, i1, r0, r1,
              acc_sh, g0, g1, t0, t1):
    c = lax.axis_index("c")
    s = lax.axis_index("s")
    wid = c * _NS + s
    isrc = (i0, i1)
    rows = (r0, r1)
    gsem = (g0, g1)
    ssem = (t0, t1)
    base0 = wid * _CPT * _CHUNK

    pltpu.sync_copy(dst2_hbm.at[pl.ds(wid * _CPT, _CPT), :], sdst)

    @pl.loop(0, _CHUNK)
    def _(i):
        for j in range(_H // 16):
            r0[i, pl.ds(j * 16, 16)] = jnp.zeros((16,), jnp.float32)

    for k in range(_RPT // _ZR):
        pltpu.sync_copy(r0, acc_sh.at[pl.ds(s * _RPT + k * _ZR, _ZR), :])
    plsc.subcore_barrier()

    for b in range(2):
        pltpu.sync_copy(srcf_hbm.at[pl.ds(base0 + b * _CHUNK, _CHUNK)],
                        isrc[b])
        pltpu.async_copy(hp_hbm.at[isrc[b]], rows[b], gsem[b]).wait()
        pltpu.async_copy(rows[b], acc_sh.at[sdst.at[b]], ssem[b], add=True)

    @pl.loop(2, _CPT, step=2)
    def _(i):
        for b in range(2):
            j = i + b
            pltpu.sync_copy(srcf_hbm.at[pl.ds(base0 + j * _CHUNK, _CHUNK)],
                            isrc[b])
            pltpu.make_async_copy(rows[b], acc_sh.at[sdst.at[j]],
                                  ssem[b]).wait()
            pltpu.async_copy(hp_hbm.at[isrc[b]], rows[b], gsem[b]).wait()
            pltpu.async_copy(rows[b], acc_sh.at[sdst.at[j]], ssem[b],
                             add=True)

    for b in range(2):
        j = _CPT - 2 + b
        pltpu.make_async_copy(rows[b], acc_sh.at[sdst.at[j]], ssem[b]).wait()

    plsc.subcore_barrier()
    pltpu.sync_copy(acc_sh.at[pl.ds(s * _RPT, _RPT), :],
                    out_hbm.at[c, pl.ds(s * _RPT, _RPT), :])


_PREC = lax.Precision.HIGHEST


def _mm1_body(p0_ref, p1_ref, x_ref, w_ref, oh_ref, od_ref):
    deg = 1.0 + p0_ref[...] + p1_ref[...]
    dinv = lax.rsqrt(deg)
    g = jnp.dot(x_ref[...], w_ref[...],
                preferred_element_type=jnp.float32, precision=_PREC)
    oh_ref[...] = g * dinv
    od_ref[...] = dinv


@jax.jit
def _mm1_call(p0, p1, x, w):
    grid = (_N // _ROWB,)
    return pl.pallas_call(
        _mm1_body,
        grid=grid,
        in_specs=[
            pl.BlockSpec((_ROWB, 1), lambda i: (i, 0)),
            pl.BlockSpec((_ROWB, 1), lambda i: (i, 0)),
            pl.BlockSpec((_ROWB, _H), lambda i: (i, 0)),
            pl.BlockSpec((_H, _H), lambda i: (0, 0)),
        ],
        out_specs=[
            pl.BlockSpec((_ROWB, _H), lambda i: (i, 0)),
            pl.BlockSpec((_ROWB, 1), lambda i: (i, 0)),
        ],
        out_shape=[
            jax.ShapeDtypeStruct((_N, _H), jnp.float32),
            jax.ShapeDtypeStruct((_N, 1), jnp.float32),
        ],
    )(p0, p1, x, w)


def _layer_body(s0_ref, s1_ref, hp_ref, d_ref, b_ref, w_ref, o_ref):
    y = d_ref[...] * (s0_ref[...] + s1_ref[...] + hp_ref[...]) + b_ref[...]
    y = jnp.maximum(y, 0.0)
    o_ref[...] = jnp.dot(y, w_ref[...],
                         preferred_element_type=jnp.float32,
                         precision=_PREC) * d_ref[...]


@jax.jit
def _layer_call(s0, s1, hp, dinv, b, w):
    grid = (_N // _ROWB,)
    return pl.pallas_call(
        _layer_body,
        grid=grid,
        in_specs=[
            pl.BlockSpec((_ROWB, _H), lambda i: (i, 0)),
            pl.BlockSpec((_ROWB, _H), lambda i: (i, 0)),
            pl.BlockSpec((_ROWB, _H), lambda i: (i, 0)),
            pl.BlockSpec((_ROWB, 1), lambda i: (i, 0)),
            pl.BlockSpec((1, _H), lambda i: (0, 0)),
            pl.BlockSpec((_H, _H), lambda i: (0, 0)),
        ],
        out_specs=pl.BlockSpec((_ROWB, _H), lambda i: (i, 0)),
        out_shape=jax.ShapeDtypeStruct((_N, _H), jnp.float32),
    )(s0, s1, hp, dinv, b, w)


def _final_body(s0_ref, s1_ref, hp_ref, d_ref, b_ref, wl_ref, bl_ref, o_ref):
    y = d_ref[...] * (s0_ref[...] + s1_ref[...] + hp_ref[...]) + b_ref[...]
    y = jnp.maximum(y, 0.0)
    o_ref[...] = jnp.dot(y, wl_ref[...],
                         preferred_element_type=jnp.float32,
                         precision=_PREC) + bl_ref[...]


@jax.jit
def _final_call(s0, s1, hp, dinv, b, wl, bl):
    grid = (_N // _ROWB,)
    return pl.pallas_call(
        _final_body,
        grid=grid,
        in_specs=[
            pl.BlockSpec((_ROWB, _H), lambda i: (i, 0)),
            pl.BlockSpec((_ROWB, _H), lambda i: (i, 0)),
            pl.BlockSpec((_ROWB, _H), lambda i: (i, 0)),
            pl.BlockSpec((_ROWB, 1), lambda i: (i, 0)),
            pl.BlockSpec((1, _H), lambda i: (0, 0)),
            pl.BlockSpec((_H, _C), lambda i: (0, 0)),
            pl.BlockSpec((1, _C), lambda i: (0, 0)),
        ],
        out_specs=pl.BlockSpec((_ROWB, _C), lambda i: (i, 0)),
        out_shape=jax.ShapeDtypeStruct((_N, _C), jnp.float32),
    )(s0, s1, hp, dinv, b, wl, bl)


@jax.jit
def kernel(x, edge_index, W1, b1, W2, b2, W3, b3, Wl, bl):
    src = edge_index[0]
    dst = edge_index[1]
    pad = _CPT * _CHUNK - _EPT
    srcf = jnp.pad(src.reshape(_NW, _EPT), ((0, 0), (0, pad))
                   ).reshape(_NW * _CPT * _CHUNK)
    dst2 = jnp.pad(dst.reshape(_NW, _EPT), ((0, 0), (0, pad)),
                   constant_values=_NP - 1).reshape(_NCH, _CHUNK)
    degp = _deg_call(dst2)
    p0 = degp[0, :_N, :1]
    p1 = degp[1, :_N, :1]
    h1, dinv = _mm1_call(p0, p1, x, W1)
    s = _agg_call(h1, srcf, dst2)
    h2 = _layer_call(s[0, :_N], s[1, :_N], h1, dinv, b1.reshape(1, _H), W2)
    s = _agg_call(h2, srcf, dst2)
    h3 = _layer_call(s[0, :_N], s[1, :_N], h2, dinv, b2.reshape(1, _H), W3)
    s = _agg_call(h3, srcf, dst2)
    out = _final_call(s[0, :_N], s[1, :_N], h3, dinv, b3.reshape(1, _H), Wl,
                      bl.reshape(1, _C))
    return out


# speedup vs baseline: 2.0319x; 1.9772x over previous
import functools

import jax
import jax.numpy as jnp
from jax import lax
from jax.experimental import pallas as pl
from jax.experimental.pallas import tpu as pltpu
from jax.experimental.pallas import tpu_sc as plsc

_N = 10000
_E = 320000
_H = 128
_C = 40
_NC = 2
_NS = 16
_NW = _NC * _NS
_CHUNK = 128
_CPT = 80
_NCH = _NW * _CPT
_EPT = _E // _NW
_NFULL = _EPT // _CHUNK
_TAIL = _EPT - _NFULL * _CHUNK
_NP = 10240
_RPT = _NP // _NS
_ZR = 128
_NBUF = 2
_ROWB = 2000

_sc_mesh = plsc.VectorSubcoreMesh(core_axis_name="c", subcore_axis_name="s")


@functools.partial(
    pl.kernel,
    out_type=jax.ShapeDtypeStruct((_NC, _NP, _H), jnp.float32),
    mesh=_sc_mesh,
    scratch_types=[
        pltpu.VMEM((_CPT, _CHUNK), jnp.int32),
        pltpu.VMEM((_CHUNK, _H), jnp.float32),
        pltpu.VMEM((_ZR, _H), jnp.float32),
        pltpu.VMEM_SHARED((_NP, _H), jnp.float32),
        pltpu.SemaphoreType.DMA,
        pltpu.SemaphoreType.DMA,
        pltpu.SemaphoreType.DMA,
        pltpu.SemaphoreType.DMA,
    ],
)
def _deg_call(dst2_hbm, out_hbm, sdst, ones_v, zbuf, acc_sh, t0, t1, t2, t3):
    c = lax.axis_index("c")
    s = lax.axis_index("s")
    wid = c * _NS + s
    ssem = (t0, t1, t2, t3)

    pltpu.sync_copy(dst2_hbm.at[pl.ds(wid * _CPT, _CPT), :], sdst)

    @pl.loop(0, _ZR)
    def _(i):
        for j in range(_H // 16):
            zbuf[i, pl.ds(j * 16, 16)] = jnp.zeros((16,), jnp.float32)

    @pl.loop(0, _CHUNK)
    def _(i):
        for j in range(_H // 16):
            ones_v[i, pl.ds(j * 16, 16)] = jnp.full((16,), 1.0, jnp.float32)

    for k in range(_RPT // _ZR):
        pltpu.sync_copy(zbuf, acc_sh.at[pl.ds(s * _RPT + k * _ZR, _ZR), :])
    plsc.subcore_barrier()

    for b in range(_NBUF):
        pltpu.async_copy(ones_v, acc_sh.at[sdst.at[b]], ssem[b], add=True)

    @pl.loop(0, _CPT - _NBUF, step=_NBUF)
    def _(i):
        for b in range(_NBUF):
            j = i + b
            pltpu.make_async_copy(ones_v, acc_sh.at[sdst.at[j]], ssem[b]).wait()
            pltpu.async_copy(ones_v, acc_sh.at[sdst.at[j + _NBUF]], ssem[b],
                             add=True)

    for b in range(_NBUF):
        j = _CPT - _NBUF + b
        pltpu.make_async_copy(ones_v, acc_sh.at[sdst.at[j]], ssem[b]).wait()

    plsc.subcore_barrier()
    pltpu.sync_copy(acc_sh.at[pl.ds(s * _RPT, _RPT), :],
                    out_hbm.at[c, pl.ds(s * _RPT, _RPT), :])


@functools.partial(
    pl.kernel,
    out_type=jax.ShapeDtypeStruct((_NC, _NP, _H), jnp.float32),
    mesh=_sc_mesh,
    scratch_types=[
        pltpu.VMEM((_CHUNK,), jnp.int32),
        pltpu.VMEM((_CHUNK,), jnp.int32),
        pltpu.VMEM((_CHUNK,), jnp.int32),
        pltpu.VMEM((_CHUNK,), jnp.int32),
        pltpu.VMEM((_CHUNK, _H), jnp.float32),
        pltpu.VMEM((_CHUNK, _H), jnp.float32),
        pltpu.VMEM((_TAIL,), jnp.int32),
        pltpu.VMEM((_TAIL,), jnp.int32),
        pltpu.VMEM((_TAIL, _H), jnp.float32),
        pltpu.VMEM_SHARED((_NP, _H), jnp.float32),
        pltpu.SemaphoreType.DMA,
        pltpu.SemaphoreType.DMA,
        pltpu.SemaphoreType.DMA,
        pltpu.SemaphoreType.DMA,
    ],
)
def _agg_call(hp_hbm, src_hbm, dst_hbm, out_hbm, s0, s1, d0, d1, r0, r1,
              src_t, dst_t, rows_t, acc_sh, g0, g1, t0, t1):
    c = lax.axis_index("c")
    s = lax.axis_index("s")
    wid = c * _NS + s
    srcv = (s0, s1)
    dstv = (d0, d1)
    rows = (r0, r1)
    gsem = (g0, g1)
    ssem = (t0, t1)
    base0 = wid * _EPT

    @pl.loop(0, _CHUNK)
    def _(i):
        for j in range(_H // 16):
            r0[i, pl.ds(j * 16, 16)] = jnp.zeros((16,), jnp.float32)

    for k in range(_RPT // _ZR):
        pltpu.sync_copy(r0, acc_sh.at[pl.ds(s * _RPT + k * _ZR, _ZR), :])
    plsc.subcore_barrier()

    for b in range(2):
        base = base0 + b * _CHUNK
        pltpu.sync_copy(src_hbm.at[pl.ds(base, _CHUNK)], srcv[b])
        pltpu.sync_copy(dst_hbm.at[pl.ds(base, _CHUNK)], dstv[b])
        pltpu.async_copy(hp_hbm.at[srcv[b]], rows[b], gsem[b]).wait()
        pltpu.async_copy(rows[b], acc_sh.at[dstv[b]], ssem[b], add=True)

    @pl.loop(2, _NFULL, step=2)
    def _(i):
        for b in range(2):
            base = base0 + (i + b) * _CHUNK
            pltpu.sync_copy(src_hbm.at[pl.ds(base, _CHUNK)], srcv[b])
            pltpu.make_async_copy(rows[b], acc_sh.at[dstv[b]],
                                  ssem[b]).wait()
            pltpu.sync_copy(dst_hbm.at[pl.ds(base, _CHUNK)], dstv[b])
            pltpu.async_copy(hp_hbm.at[srcv[b]], rows[b], gsem[b]).wait()
            pltpu.async_copy(rows[b], acc_sh.at[dstv[b]], ssem[b], add=True)

    for b in range(2):
        pltpu.make_async_copy(rows[b], acc_sh.at[dstv[b]], ssem[b]).wait()

    tbase = base0 + _NFULL * _CHUNK
    pltpu.sync_copy(src_hbm.at[pl.ds(tbase, _TAIL)], src_t)
    pltpu.sync_copy(dst_hbm.at[pl.ds(tbase, _TAIL)], dst_t)
    pltpu.async_copy(hp_hbm.at[src_t], rows_t, gsem[0]).wait()
    pltpu.sync_copy(rows_t, acc_sh.at[dst_t], add=True)

    plsc.subcore_barrier()
    pltpu.sync_copy(acc_sh.at[pl.ds(s * _RPT, _RPT), :],
                    out_hbm.at[c, pl.ds(s * _RPT, _RPT), :])


_PREC = lax.Precision.HIGHEST


def _mm1_body(p0_ref, p1_ref, x_ref, w_ref, oh_ref, od_ref):
    deg = 1.0 + p0_ref[...] + p1_ref[...]
    dinv = lax.rsqrt(deg)
    g = jnp.dot(x_ref[...], w_ref[...],
                preferred_element_type=jnp.float32, precision=_PREC)
    oh_ref[...] = g * dinv
    od_ref[...] = dinv


@jax.jit
def _mm1_call(p0, p1, x, w):
    grid = (_N // _ROWB,)
    return pl.pallas_call(
        _mm1_body,
        grid=grid,
        in_specs=[
            pl.BlockSpec((_ROWB, 1), lambda i: (i, 0)),
            pl.BlockSpec((_ROWB, 1), lambda i: (i, 0)),
            pl.BlockSpec((_ROWB, _H), lambda i: (i, 0)),
            pl.BlockSpec((_H, _H), lambda i: (0, 0)),
        ],
        out_specs=[
            pl.BlockSpec((_ROWB, _H), lambda i: (i, 0)),
            pl.BlockSpec((_ROWB, 1), lambda i: (i, 0)),
        ],
        out_shape=[
            jax.ShapeDtypeStruct((_N, _H), jnp.float32),
            jax.ShapeDtypeStruct((_N, 1), jnp.float32),
        ],
    )(p0, p1, x, w)


def _layer_body(s0_ref, s1_ref, hp_ref, d_ref, b_ref, w_ref, o_ref):
    y = d_ref[...] * (s0_ref[...] + s1_ref[...] + hp_ref[...]) + b_ref[...]
    y = jnp.maximum(y, 0.0)
    o_ref[...] = jnp.dot(y, w_ref[...],
                         preferred_element_type=jnp.float32,
                         precision=_PREC) * d_ref[...]


@jax.jit
def _layer_call(s0, s1, hp, dinv, b, w):
    grid = (_N // _ROWB,)
    return pl.pallas_call(
        _layer_body,
        grid=grid,
        in_specs=[
            pl.BlockSpec((_ROWB, _H), lambda i: (i, 0)),
            pl.BlockSpec((_ROWB, _H), lambda i: (i, 0)),
            pl.BlockSpec((_ROWB, _H), lambda i: (i, 0)),
            pl.BlockSpec((_ROWB, 1), lambda i: (i, 0)),
            pl.BlockSpec((1, _H), lambda i: (0, 0)),
            pl.BlockSpec((_H, _H), lambda i: (0, 0)),
        ],
        out_specs=pl.BlockSpec((_ROWB, _H), lambda i: (i, 0)),
        out_shape=jax.ShapeDtypeStruct((_N, _H), jnp.float32),
    )(s0, s1, hp, dinv, b, w)


def _final_body(s0_ref, s1_ref, hp_ref, d_ref, b_ref, wl_ref, bl_ref, o_ref):
    y = d_ref[...] * (s0_ref[...] + s1_ref[...] + hp_ref[...]) + b_ref[...]
    y = jnp.maximum(y, 0.0)
    o_ref[...] = jnp.dot(y, wl_ref[...],
                         preferred_element_type=jnp.float32,
                         precision=_PREC) + bl_ref[...]


@jax.jit
def _final_call(s0, s1, hp, dinv, b, wl, bl):
    grid = (_N // _ROWB,)
    return pl.pallas_call(
        _final_body,
        grid=grid,
        in_specs=[
            pl.BlockSpec((_ROWB, _H), lambda i: (i, 0)),
            pl.BlockSpec((_ROWB, _H), lambda i: (i, 0)),
            pl.BlockSpec((_ROWB, _H), lambda i: (i, 0)),
            pl.BlockSpec((_ROWB, 1), lambda i: (i, 0)),
            pl.BlockSpec((1, _H), lambda i: (0, 0)),
            pl.BlockSpec((_H, _C), lambda i: (0, 0)),
            pl.BlockSpec((1, _C), lambda i: (0, 0)),
        ],
        out_specs=pl.BlockSpec((_ROWB, _C), lambda i: (i, 0)),
        out_shape=jax.ShapeDtypeStruct((_N, _C), jnp.float32),
    )(s0, s1, hp, dinv, b, wl, bl)


@jax.jit
def kernel(x, edge_index, W1, b1, W2, b2, W3, b3, Wl, bl):
    src = edge_index[0]
    dst = edge_index[1]
    pad = _CPT * _CHUNK - _EPT
    dst2 = jnp.pad(dst.reshape(_NW, _EPT), ((0, 0), (0, pad)),
                   constant_values=_NP - 1).reshape(_NCH, _CHUNK)
    degp = _deg_call(dst2)
    p0 = degp[0, :_N, :1]
    p1 = degp[1, :_N, :1]
    h1, dinv = _mm1_call(p0, p1, x, W1)
    s = _agg_call(h1, src, dst)
    h2 = _layer_call(s[0, :_N], s[1, :_N], h1, dinv, b1.reshape(1, _H), W2)
    s = _agg_call(h2, src, dst)
    h3 = _layer_call(s[0, :_N], s[1, :_N], h2, dinv, b2.reshape(1, _H), W3)
    s = _agg_call(h3, src, dst)
    out = _final_call(s[0, :_N], s[1, :_N], h3, dinv, b3.reshape(1, _H), Wl,
                      bl.reshape(1, _C))
    return out


# speedup vs baseline: 2.7430x; 1.3500x over previous
import functools

import jax
import jax.numpy as jnp
from jax import lax
from jax.experimental import pallas as pl
from jax.experimental.pallas import tpu as pltpu
from jax.experimental.pallas import tpu_sc as plsc

_N = 10000
_E = 320000
_H = 128
_C = 40
_NC = 2
_NS = 16
_NW = _NC * _NS
_CHUNK = 128
_CPT = 80
_NCH = _NW * _CPT
_EPT = _E // _NW
_NFULL = _EPT // _CHUNK
_TAIL = _EPT - _NFULL * _CHUNK
_NP = 10240
_RPT = _NP // _NS
_ZR = 128
_NBUF = 2
_ROWB = 2000

_sc_mesh = plsc.VectorSubcoreMesh(core_axis_name="c", subcore_axis_name="s")


@functools.partial(
    pl.kernel,
    out_type=jax.ShapeDtypeStruct((_NC, _NP, _H), jnp.float32),
    mesh=_sc_mesh,
    scratch_types=[
        pltpu.VMEM((_CPT, _CHUNK), jnp.int32),
        pltpu.VMEM((_CHUNK, _H), jnp.float32),
        pltpu.VMEM((_ZR, _H), jnp.float32),
        pltpu.VMEM_SHARED((_NP, _H), jnp.float32),
        pltpu.SemaphoreType.DMA,
        pltpu.SemaphoreType.DMA,
        pltpu.SemaphoreType.DMA,
        pltpu.SemaphoreType.DMA,
    ],
)
def _deg_call(dst2_hbm, out_hbm, sdst, ones_v, zbuf, acc_sh, t0, t1, t2, t3):
    c = lax.axis_index("c")
    s = lax.axis_index("s")
    wid = c * _NS + s
    ssem = (t0, t1, t2, t3)

    pltpu.sync_copy(dst2_hbm.at[pl.ds(wid * _CPT, _CPT), :], sdst)

    @pl.loop(0, _ZR)
    def _(i):
        for j in range(_H // 16):
            zbuf[i, pl.ds(j * 16, 16)] = jnp.zeros((16,), jnp.float32)

    @pl.loop(0, _CHUNK)
    def _(i):
        for j in range(_H // 16):
            ones_v[i, pl.ds(j * 16, 16)] = jnp.full((16,), 1.0, jnp.float32)

    for k in range(_RPT // _ZR):
        pltpu.sync_copy(zbuf, acc_sh.at[pl.ds(s * _RPT + k * _ZR, _ZR), :])
    plsc.subcore_barrier()

    for b in range(_NBUF):
        pltpu.async_copy(ones_v, acc_sh.at[sdst.at[b]], ssem[b], add=True)

    @pl.loop(0, _CPT - _NBUF, step=_NBUF)
    def _(i):
        for b in range(_NBUF):
            j = i + b
            pltpu.make_async_copy(ones_v, acc_sh.at[sdst.at[j]], ssem[b]).wait()
            pltpu.async_copy(ones_v, acc_sh.at[sdst.at[j + _NBUF]], ssem[b],
                             add=True)

    for b in range(_NBUF):
        j = _CPT - _NBUF + b
        pltpu.make_async_copy(ones_v, acc_sh.at[sdst.at[j]], ssem[b]).wait()

    plsc.subcore_barrier()
    pltpu.sync_copy(acc_sh.at[pl.ds(s * _RPT, _RPT), :],
                    out_hbm.at[c, pl.ds(s * _RPT, _RPT), :])


@functools.partial(
    pl.kernel,
    out_type=jax.ShapeDtypeStruct((_NC, _NP, _H), jnp.float32),
    mesh=_sc_mesh,
    scratch_types=[
        pltpu.VMEM((_CHUNK,), jnp.int32),
        pltpu.VMEM((_CHUNK,), jnp.int32),
        pltpu.VMEM((_CHUNK,), jnp.int32),
        pltpu.VMEM((_CHUNK,), jnp.int32),
        pltpu.VMEM((_CHUNK,), jnp.int32),
        pltpu.VMEM((_CHUNK,), jnp.int32),
        pltpu.VMEM((_CHUNK,), jnp.int32),
        pltpu.VMEM((_CHUNK,), jnp.int32),
        pltpu.VMEM((_CHUNK, _H), jnp.float32),
        pltpu.VMEM((_CHUNK, _H), jnp.float32),
        pltpu.VMEM((_TAIL,), jnp.int32),
        pltpu.VMEM((_TAIL,), jnp.int32),
        pltpu.VMEM((_TAIL, _H), jnp.float32),
        pltpu.VMEM_SHARED((_NP, _H), jnp.float32),
        pltpu.SemaphoreType.DMA,
        pltpu.SemaphoreType.DMA,
        pltpu.SemaphoreType.DMA,
        pltpu.SemaphoreType.DMA,
        pltpu.SemaphoreType.DMA,
        pltpu.SemaphoreType.DMA,
        pltpu.SemaphoreType.DMA,
        pltpu.SemaphoreType.DMA,
        pltpu.SemaphoreType.DMA,
        pltpu.SemaphoreType.DMA,
        pltpu.SemaphoreType.DMA,
        pltpu.SemaphoreType.DMA,
    ],
)
def _agg_call(hp_hbm, src_hbm, dst_hbm, out_hbm,
              sv0, sv1, sv2, sv3, dv0, dv1, dv2, dv3, r0, r1,
              src_t, dst_t, rows_t, acc_sh,
              p0, p1, p2, p3, q0, q1, q2, q3, g0, g1, t0, t1):
    c = lax.axis_index("c")
    s = lax.axis_index("s")
    wid = c * _NS + s
    srcv = (sv0, sv1, sv2, sv3)
    dstv = (dv0, dv1, dv2, dv3)
    psem = (p0, p1, p2, p3)
    qsem = (q0, q1, q2, q3)
    rows = (r0, r1)
    gsem = (g0, g1)
    ssem = (t0, t1)
    base0 = wid * _EPT

    def fire_idx(j, slot):
        pltpu.async_copy(src_hbm.at[pl.ds(base0 + j * _CHUNK, _CHUNK)],
                         srcv[slot], psem[slot])
        pltpu.async_copy(dst_hbm.at[pl.ds(base0 + j * _CHUNK, _CHUNK)],
                         dstv[slot], qsem[slot])

    def wait_idx(j, slot):
        pltpu.make_async_copy(src_hbm.at[pl.ds(base0 + j * _CHUNK, _CHUNK)],
                              srcv[slot], psem[slot]).wait()
        pltpu.make_async_copy(dst_hbm.at[pl.ds(base0 + j * _CHUNK, _CHUNK)],
                              dstv[slot], qsem[slot]).wait()

    @pl.loop(0, _CHUNK)
    def _(i):
        for j in range(_H // 16):
            r0[i, pl.ds(j * 16, 16)] = jnp.zeros((16,), jnp.float32)

    for k in range(_RPT // _ZR):
        pltpu.sync_copy(r0, acc_sh.at[pl.ds(s * _RPT + k * _ZR, _ZR), :])

    for slot in range(4):
        fire_idx(slot, slot)
    plsc.subcore_barrier()

    for b in range(2):
        wait_idx(b, b)
        pltpu.async_copy(hp_hbm.at[srcv[b]], rows[b], gsem[b]).wait()
        pltpu.async_copy(rows[b], acc_sh.at[dstv[b]], ssem[b], add=True)

    @pl.loop(2, _NFULL, step=4)
    def _(i):
        for u in range(4):
            j = i + u
            slot = (2 + u) % 4
            b = u % 2
            pltpu.make_async_copy(rows[b], acc_sh.at[dstv[(slot + 2) % 4]],
                                  ssem[b]).wait()
            fire_idx(j + 2, (slot + 2) % 4)
            wait_idx(j, slot)
            pltpu.async_copy(hp_hbm.at[srcv[slot]], rows[b], gsem[b]).wait()
            pltpu.async_copy(rows[b], acc_sh.at[dstv[slot]], ssem[b],
                             add=True)

    for b in range(2):
        pltpu.make_async_copy(rows[b], acc_sh.at[dstv[b]], ssem[b]).wait()
    wait_idx(_NFULL, 2)
    wait_idx(_NFULL + 1, 3)

    tbase = base0 + _NFULL * _CHUNK
    pltpu.sync_copy(src_hbm.at[pl.ds(tbase, _TAIL)], src_t)
    pltpu.sync_copy(dst_hbm.at[pl.ds(tbase, _TAIL)], dst_t)
    pltpu.async_copy(hp_hbm.at[src_t], rows_t, gsem[0]).wait()
    pltpu.sync_copy(rows_t, acc_sh.at[dst_t], add=True)

    plsc.subcore_barrier()
    pltpu.sync_copy(acc_sh.at[pl.ds(s * _RPT, _RPT), :],
                    out_hbm.at[c, pl.ds(s * _RPT, _RPT), :])


_PREC = lax.Precision.HIGHEST


def _mm1_body(p0_ref, p1_ref, x_ref, w_ref, oh_ref, od_ref):
    deg = 1.0 + p0_ref[...] + p1_ref[...]
    dinv = lax.rsqrt(deg)
    g = jnp.dot(x_ref[...], w_ref[...],
                preferred_element_type=jnp.float32, precision=_PREC)
    oh_ref[...] = g * dinv
    od_ref[...] = dinv


@jax.jit
def _mm1_call(p0, p1, x, w):
    grid = (_N // _ROWB,)
    return pl.pallas_call(
        _mm1_body,
        grid=grid,
        in_specs=[
            pl.BlockSpec((_ROWB, 1), lambda i: (i, 0)),
            pl.BlockSpec((_ROWB, 1), lambda i: (i, 0)),
            pl.BlockSpec((_ROWB, _H), lambda i: (i, 0)),
            pl.BlockSpec((_H, _H), lambda i: (0, 0)),
        ],
        out_specs=[
            pl.BlockSpec((_ROWB, _H), lambda i: (i, 0)),
            pl.BlockSpec((_ROWB, 1), lambda i: (i, 0)),
        ],
        out_shape=[
            jax.ShapeDtypeStruct((_N, _H), jnp.float32),
            jax.ShapeDtypeStruct((_N, 1), jnp.float32),
        ],
    )(p0, p1, x, w)


def _layer_body(s0_ref, s1_ref, hp_ref, d_ref, b_ref, w_ref, o_ref):
    y = d_ref[...] * (s0_ref[...] + s1_ref[...] + hp_ref[...]) + b_ref[...]
    y = jnp.maximum(y, 0.0)
    o_ref[...] = jnp.dot(y, w_ref[...],
                         preferred_element_type=jnp.float32,
                         precision=_PREC) * d_ref[...]


@jax.jit
def _layer_call(s0, s1, hp, dinv, b, w):
    grid = (_N // _ROWB,)
    return pl.pallas_call(
        _layer_body,
        grid=grid,
        in_specs=[
            pl.BlockSpec((_ROWB, _H), lambda i: (i, 0)),
            pl.BlockSpec((_ROWB, _H), lambda i: (i, 0)),
            pl.BlockSpec((_ROWB, _H), lambda i: (i, 0)),
            pl.BlockSpec((_ROWB, 1), lambda i: (i, 0)),
            pl.BlockSpec((1, _H), lambda i: (0, 0)),
            pl.BlockSpec((_H, _H), lambda i: (0, 0)),
        ],
        out_specs=pl.BlockSpec((_ROWB, _H), lambda i: (i, 0)),
        out_shape=jax.ShapeDtypeStruct((_N, _H), jnp.float32),
    )(s0, s1, hp, dinv, b, w)


def _final_body(s0_ref, s1_ref, hp_ref, d_ref, b_ref, wl_ref, bl_ref, o_ref):
    y = d_ref[...] * (s0_ref[...] + s1_ref[...] + hp_ref[...]) + b_ref[...]
    y = jnp.maximum(y, 0.0)
    o_ref[...] = jnp.dot(y, wl_ref[...],
                         preferred_element_type=jnp.float32,
                         precision=_PREC) + bl_ref[...]


@jax.jit
def _final_call(s0, s1, hp, dinv, b, wl, bl):
    grid = (_N // _ROWB,)
    return pl.pallas_call(
        _final_body,
        grid=grid,
        in_specs=[
            pl.BlockSpec((_ROWB, _H), lambda i: (i, 0)),
            pl.BlockSpec((_ROWB, _H), lambda i: (i, 0)),
            pl.BlockSpec((_ROWB, _H), lambda i: (i, 0)),
            pl.BlockSpec((_ROWB, 1), lambda i: (i, 0)),
            pl.BlockSpec((1, _H), lambda i: (0, 0)),
            pl.BlockSpec((_H, _C), lambda i: (0, 0)),
            pl.BlockSpec((1, _C), lambda i: (0, 0)),
        ],
        out_specs=pl.BlockSpec((_ROWB, _C), lambda i: (i, 0)),
        out_shape=jax.ShapeDtypeStruct((_N, _C), jnp.float32),
    )(s0, s1, hp, dinv, b, wl, bl)


@jax.jit
def kernel(x, edge_index, W1, b1, W2, b2, W3, b3, Wl, bl):
    srcr = edge_index[0]
    dstr = edge_index[1]
    src = jnp.concatenate([srcr, jnp.zeros((2 * _CHUNK,), jnp.int32)])
    dst = jnp.concatenate([dstr, jnp.zeros((2 * _CHUNK,), jnp.int32)])
    pad = _CPT * _CHUNK - _EPT
    dst2 = jnp.pad(dstr.reshape(_NW, _EPT), ((0, 0), (0, pad)),
                   constant_values=_NP - 1).reshape(_NCH, _CHUNK)
    degp = _deg_call(dst2)
    p0 = degp[0, :_N, :1]
    p1 = degp[1, :_N, :1]
    h1, dinv = _mm1_call(p0, p1, x, W1)
    s = _agg_call(h1, src, dst)
    h2 = _layer_call(s[0, :_N], s[1, :_N], h1, dinv, b1.reshape(1, _H), W2)
    s = _agg_call(h2, src, dst)
    h3 = _layer_call(s[0, :_N], s[1, :_N], h2, dinv, b2.reshape(1, _H), W3)
    s = _agg_call(h3, src, dst)
    out = _final_call(s[0, :_N], s[1, :_N], h3, dinv, b3.reshape(1, _H), Wl,
                      bl.reshape(1, _C))
    return out


# speedup vs baseline: 2.7457x; 1.0010x over previous
import functools

import jax
import jax.numpy as jnp
from jax import lax
from jax.experimental import pallas as pl
from jax.experimental.pallas import tpu as pltpu
from jax.experimental.pallas import tpu_sc as plsc

_N = 10000
_E = 320000
_H = 128
_C = 40
_NC = 2
_NS = 16
_NW = _NC * _NS
_CHUNK = 128
_CPT = 80
_NCH = _NW * _CPT
_EPT = _E // _NW
_NFULL = _EPT // _CHUNK
_TAIL = _EPT - _NFULL * _CHUNK
_NP = 10240
_RPT = _NP // _NS
_ZR = 128
_NBUF = 2
_ROWB = 2000

_sc_mesh = plsc.VectorSubcoreMesh(core_axis_name="c", subcore_axis_name="s")


@functools.partial(
    pl.kernel,
    out_type=jax.ShapeDtypeStruct((_NC, _NP, _H), jnp.float32),
    mesh=_sc_mesh,
    scratch_types=[
        pltpu.VMEM((_CHUNK,), jnp.int32),
        pltpu.VMEM((_CHUNK,), jnp.int32),
        pltpu.VMEM((_CHUNK,), jnp.int32),
        pltpu.VMEM((_CHUNK,), jnp.int32),
        pltpu.VMEM((_CHUNK, _H), jnp.float32),
        pltpu.VMEM((_ZR, _H), jnp.float32),
        pltpu.VMEM_SHARED((_NP, _H), jnp.float32),
        pltpu.SemaphoreType.DMA,
        pltpu.SemaphoreType.DMA,
        pltpu.SemaphoreType.DMA,
        pltpu.SemaphoreType.DMA,
        pltpu.SemaphoreType.DMA,
        pltpu.SemaphoreType.DMA,
    ],
)
def _deg_call(dstf_hbm, out_hbm, dv0, dv1, dv2, dv3, ones_v, zbuf, acc_sh,
              q0, q1, q2, q3, t0, t1):
    c = lax.axis_index("c")
    s = lax.axis_index("s")
    wid = c * _NS + s
    dstv = (dv0, dv1, dv2, dv3)
    qsem = (q0, q1, q2, q3)
    ssem = (t0, t1)
    base0 = wid * _CPT * _CHUNK

    def fire_idx(j, slot):
        pltpu.async_copy(dstf_hbm.at[pl.ds(base0 + j * _CHUNK, _CHUNK)],
                         dstv[slot], qsem[slot])

    def wait_idx(j, slot):
        pltpu.make_async_copy(dstf_hbm.at[pl.ds(base0 + j * _CHUNK, _CHUNK)],
                              dstv[slot], qsem[slot]).wait()

    @pl.loop(0, _ZR)
    def _(i):
        for j in range(_H // 16):
            zbuf[i, pl.ds(j * 16, 16)] = jnp.zeros((16,), jnp.float32)

    @pl.loop(0, _CHUNK)
    def _(i):
        for j in range(_H // 16):
            ones_v[i, pl.ds(j * 16, 16)] = jnp.full((16,), 1.0, jnp.float32)

    for k in range(_RPT // _ZR):
        pltpu.sync_copy(zbuf, acc_sh.at[pl.ds(s * _RPT + k * _ZR, _ZR), :])

    for slot in range(4):
        fire_idx(slot, slot)
    plsc.subcore_barrier()

    for b in range(2):
        wait_idx(b, b)
        pltpu.async_copy(ones_v, acc_sh.at[dstv[b]], ssem[b], add=True)

    @pl.loop(2, _CPT, step=4)
    def _(i):
        for u in range(4):
            j = i + u
            slot = (2 + u) % 4
            b = u % 2
            pltpu.make_async_copy(ones_v, acc_sh.at[dstv[(slot + 2) % 4]],
                                  ssem[b]).wait()
            fire_idx(j + 2, (slot + 2) % 4)
            wait_idx(j, slot)
            pltpu.async_copy(ones_v, acc_sh.at[dstv[slot]], ssem[b],
                             add=True)

    for b in range(2):
        pltpu.make_async_copy(ones_v, acc_sh.at[dstv[b]], ssem[b]).wait()
    wait_idx(_CPT, 2)
    wait_idx(_CPT + 1, 3)

    plsc.subcore_barrier()
    pltpu.sync_copy(acc_sh.at[pl.ds(s * _RPT, _RPT), :],
                    out_hbm.at[c, pl.ds(s * _RPT, _RPT), :])


@functools.partial(
    pl.kernel,
    out_type=jax.ShapeDtypeStruct((_NC, _NP, _H), jnp.float32),
    mesh=_sc_mesh,
    scratch_types=[
        pltpu.VMEM((_CHUNK,), jnp.int32),
        pltpu.VMEM((_CHUNK,), jnp.int32),
        pltpu.VMEM((_CHUNK,), jnp.int32),
        pltpu.VMEM((_CHUNK,), jnp.int32),
        pltpu.VMEM((_CHUNK,), jnp.int32),
        pltpu.VMEM((_CHUNK,), jnp.int32),
        pltpu.VMEM((_CHUNK,), jnp.int32),
        pltpu.VMEM((_CHUNK,), jnp.int32),
        pltpu.VMEM((_CHUNK, _H), jnp.float32),
        pltpu.VMEM((_CHUNK, _H), jnp.float32),
        pltpu.VMEM((_TAIL,), jnp.int32),
        pltpu.VMEM((_TAIL,), jnp.int32),
        pltpu.VMEM((_TAIL, _H), jnp.float32),
        pltpu.VMEM_SHARED((_NP, _H), jnp.float32),
        pltpu.SemaphoreType.DMA,
        pltpu.SemaphoreType.DMA,
        pltpu.SemaphoreType.DMA,
        pltpu.SemaphoreType.DMA,
        pltpu.SemaphoreType.DMA,
        pltpu.SemaphoreType.DMA,
        pltpu.SemaphoreType.DMA,
        pltpu.SemaphoreType.DMA,
        pltpu.SemaphoreType.DMA,
        pltpu.SemaphoreType.DMA,
        pltpu.SemaphoreType.DMA,
        pltpu.SemaphoreType.DMA,
    ],
)
def _agg_call(hp_hbm, src_hbm, dst_hbm, out_hbm,
              sv0, sv1, sv2, sv3, dv0, dv1, dv2, dv3, r0, r1,
              src_t, dst_t, rows_t, acc_sh,
              p0, p1, p2, p3, q0, q1, q2, q3, g0, g1, t0, t1):
    c = lax.axis_index("c")
    s = lax.axis_index("s")
    wid = c * _NS + s
    srcv = (sv0, sv1, sv2, sv3)
    dstv = (dv0, dv1, dv2, dv3)
    psem = (p0, p1, p2, p3)
    qsem = (q0, q1, q2, q3)
    rows = (r0, r1)
    gsem = (g0, g1)
    ssem = (t0, t1)
    base0 = wid * _EPT

    def fire_idx(j, slot):
        pltpu.async_copy(src_hbm.at[pl.ds(base0 + j * _CHUNK, _CHUNK)],
                         srcv[slot], psem[slot])
        pltpu.async_copy(dst_hbm.at[pl.ds(base0 + j * _CHUNK, _CHUNK)],
                         dstv[slot], qsem[slot])

    def wait_idx(j, slot):
        pltpu.make_async_copy(src_hbm.at[pl.ds(base0 + j * _CHUNK, _CHUNK)],
                              srcv[slot], psem[slot]).wait()
        pltpu.make_async_copy(dst_hbm.at[pl.ds(base0 + j * _CHUNK, _CHUNK)],
                              dstv[slot], qsem[slot]).wait()

    @pl.loop(0, _CHUNK)
    def _(i):
        for j in range(_H // 16):
            r0[i, pl.ds(j * 16, 16)] = jnp.zeros((16,), jnp.float32)

    for k in range(_RPT // _ZR):
        pltpu.sync_copy(r0, acc_sh.at[pl.ds(s * _RPT + k * _ZR, _ZR), :])

    for slot in range(4):
        fire_idx(slot, slot)
    plsc.subcore_barrier()

    for b in range(2):
        wait_idx(b, b)
        pltpu.async_copy(hp_hbm.at[srcv[b]], rows[b], gsem[b]).wait()
        pltpu.async_copy(rows[b], acc_sh.at[dstv[b]], ssem[b], add=True)

    @pl.loop(2, _NFULL, step=4)
    def _(i):
        for u in range(4):
            j = i + u
            slot = (2 + u) % 4
            b = u % 2
            pltpu.make_async_copy(rows[b], acc_sh.at[dstv[(slot + 2) % 4]],
                                  ssem[b]).wait()
            fire_idx(j + 2, (slot + 2) % 4)
            wait_idx(j, slot)
            pltpu.async_copy(hp_hbm.at[srcv[slot]], rows[b], gsem[b]).wait()
            pltpu.async_copy(rows[b], acc_sh.at[dstv[slot]], ssem[b],
                             add=True)

    for b in range(2):
        pltpu.make_async_copy(rows[b], acc_sh.at[dstv[b]], ssem[b]).wait()
    wait_idx(_NFULL, 2)
    wait_idx(_NFULL + 1, 3)

    tbase = base0 + _NFULL * _CHUNK
    pltpu.sync_copy(src_hbm.at[pl.ds(tbase, _TAIL)], src_t)
    pltpu.sync_copy(dst_hbm.at[pl.ds(tbase, _TAIL)], dst_t)
    pltpu.async_copy(hp_hbm.at[src_t], rows_t, gsem[0]).wait()
    pltpu.sync_copy(rows_t, acc_sh.at[dst_t], add=True)

    plsc.subcore_barrier()
    pltpu.sync_copy(acc_sh.at[pl.ds(s * _RPT, _RPT), :],
                    out_hbm.at[c, pl.ds(s * _RPT, _RPT), :])


_PREC = lax.Precision.HIGHEST


def _mm1_body(p0_ref, p1_ref, x_ref, w_ref, oh_ref, od_ref):
    deg = 1.0 + p0_ref[...] + p1_ref[...]
    dinv = lax.rsqrt(deg)
    g = jnp.dot(x_ref[...], w_ref[...],
                preferred_element_type=jnp.float32, precision=_PREC)
    oh_ref[...] = g * dinv
    od_ref[...] = dinv


@jax.jit
def _mm1_call(p0, p1, x, w):
    grid = (_N // _ROWB,)
    return pl.pallas_call(
        _mm1_body,
        grid=grid,
        in_specs=[
            pl.BlockSpec((_ROWB, 1), lambda i: (i, 0)),
            pl.BlockSpec((_ROWB, 1), lambda i: (i, 0)),
            pl.BlockSpec((_ROWB, _H), lambda i: (i, 0)),
            pl.BlockSpec((_H, _H), lambda i: (0, 0)),
        ],
        out_specs=[
            pl.BlockSpec((_ROWB, _H), lambda i: (i, 0)),
            pl.BlockSpec((_ROWB, 1), lambda i: (i, 0)),
        ],
        out_shape=[
            jax.ShapeDtypeStruct((_N, _H), jnp.float32),
            jax.ShapeDtypeStruct((_N, 1), jnp.float32),
        ],
    )(p0, p1, x, w)


def _layer_body(s0_ref, s1_ref, hp_ref, d_ref, b_ref, w_ref, o_ref):
    y = d_ref[...] * (s0_ref[...] + s1_ref[...] + hp_ref[...]) + b_ref[...]
    y = jnp.maximum(y, 0.0)
    o_ref[...] = jnp.dot(y, w_ref[...],
                         preferred_element_type=jnp.float32,
                         precision=_PREC) * d_ref[...]


@jax.jit
def _layer_call(s0, s1, hp, dinv, b, w):
    grid = (_N // _ROWB,)
    return pl.pallas_call(
        _layer_body,
        grid=grid,
        in_specs=[
            pl.BlockSpec((_ROWB, _H), lambda i: (i, 0)),
            pl.BlockSpec((_ROWB, _H), lambda i: (i, 0)),
            pl.BlockSpec((_ROWB, _H), lambda i: (i, 0)),
            pl.BlockSpec((_ROWB, 1), lambda i: (i, 0)),
            pl.BlockSpec((1, _H), lambda i: (0, 0)),
            pl.BlockSpec((_H, _H), lambda i: (0, 0)),
        ],
        out_specs=pl.BlockSpec((_ROWB, _H), lambda i: (i, 0)),
        out_shape=jax.ShapeDtypeStruct((_N, _H), jnp.float32),
    )(s0, s1, hp, dinv, b, w)


def _final_body(s0_ref, s1_ref, hp_ref, d_ref, b_ref, wl_ref, bl_ref, o_ref):
    y = d_ref[...] * (s0_ref[...] + s1_ref[...] + hp_ref[...]) + b_ref[...]
    y = jnp.maximum(y, 0.0)
    o_ref[...] = jnp.dot(y, wl_ref[...],
                         preferred_element_type=jnp.float32,
                         precision=_PREC) + bl_ref[...]


@jax.jit
def _final_call(s0, s1, hp, dinv, b, wl, bl):
    grid = (_N // _ROWB,)
    return pl.pallas_call(
        _final_body,
        grid=grid,
        in_specs=[
            pl.BlockSpec((_ROWB, _H), lambda i: (i, 0)),
            pl.BlockSpec((_ROWB, _H), lambda i: (i, 0)),
            pl.BlockSpec((_ROWB, _H), lambda i: (i, 0)),
            pl.BlockSpec((_ROWB, 1), lambda i: (i, 0)),
            pl.BlockSpec((1, _H), lambda i: (0, 0)),
            pl.BlockSpec((_H, _C), lambda i: (0, 0)),
            pl.BlockSpec((1, _C), lambda i: (0, 0)),
        ],
        out_specs=pl.BlockSpec((_ROWB, _C), lambda i: (i, 0)),
        out_shape=jax.ShapeDtypeStruct((_N, _C), jnp.float32),
    )(s0, s1, hp, dinv, b, wl, bl)


@jax.jit
def kernel(x, edge_index, W1, b1, W2, b2, W3, b3, Wl, bl):
    srcr = edge_index[0]
    dstr = edge_index[1]
    src = jnp.concatenate([srcr, jnp.zeros((2 * _CHUNK,), jnp.int32)])
    dst = jnp.concatenate([dstr, jnp.zeros((2 * _CHUNK,), jnp.int32)])
    pad = _CPT * _CHUNK - _EPT
    dstf = jnp.concatenate([
        jnp.pad(dstr.reshape(_NW, _EPT), ((0, 0), (0, pad)),
                constant_values=_NP - 1).reshape(_NW * _CPT * _CHUNK),
        jnp.zeros((2 * _CHUNK,), jnp.int32)])
    degp = _deg_call(dstf)
    p0 = degp[0, :_N, :1]
    p1 = degp[1, :_N, :1]
    h1, dinv = _mm1_call(p0, p1, x, W1)
    s = _agg_call(h1, src, dst)
    h2 = _layer_call(s[0, :_N], s[1, :_N], h1, dinv, b1.reshape(1, _H), W2)
    s = _agg_call(h2, src, dst)
    h3 = _layer_call(s[0, :_N], s[1, :_N], h2, dinv, b2.reshape(1, _H), W3)
    s = _agg_call(h3, src, dst)
    out = _final_call(s[0, :_N], s[1, :_N], h3, dinv, b3.reshape(1, _H), Wl,
                      bl.reshape(1, _C))
    return out


# speedup vs baseline: 2.7566x; 1.0040x over previous
import functools

import jax
import jax.numpy as jnp
from jax import lax
from jax.experimental import pallas as pl
from jax.experimental.pallas import tpu as pltpu
from jax.experimental.pallas import tpu_sc as plsc

_N = 10000
_E = 320000
_H = 128
_C = 40
_NC = 2
_NS = 16
_NW = _NC * _NS
_CHUNK = 128
_CPT = 80
_NCH = _NW * _CPT
_EPT = _E // _NW
_NFULL = _EPT // _CHUNK
_TAIL = _EPT - _NFULL * _CHUNK
_NP = 10240
_RPT = _NP // _NS
_ZR = 128
_NBUF = 2
_ROWB = 2000

_sc_mesh = plsc.VectorSubcoreMesh(core_axis_name="c", subcore_axis_name="s")


@functools.partial(
    pl.kernel,
    out_type=jax.ShapeDtypeStruct((_NC, _NP, _H), jnp.float32),
    mesh=_sc_mesh,
    scratch_types=[
        pltpu.VMEM((_CHUNK,), jnp.int32),
        pltpu.VMEM((_CHUNK,), jnp.int32),
        pltpu.VMEM((_CHUNK,), jnp.int32),
        pltpu.VMEM((_CHUNK,), jnp.int32),
        pltpu.VMEM((_CHUNK, _H), jnp.float32),
        pltpu.VMEM((_ZR, _H), jnp.float32),
        pltpu.VMEM_SHARED((_NP, _H), jnp.float32),
        pltpu.SemaphoreType.DMA,
        pltpu.SemaphoreType.DMA,
        pltpu.SemaphoreType.DMA,
        pltpu.SemaphoreType.DMA,
        pltpu.SemaphoreType.DMA,
        pltpu.SemaphoreType.DMA,
    ],
)
def _deg_call(dstf_hbm, out_hbm, dv0, dv1, dv2, dv3, ones_v, zbuf, acc_sh,
              q0, q1, q2, q3, t0, t1):
    c = lax.axis_index("c")
    s = lax.axis_index("s")
    wid = c * _NS + s
    dstv = (dv0, dv1, dv2, dv3)
    qsem = (q0, q1, q2, q3)
    ssem = (t0, t1)
    base0 = wid * _CPT * _CHUNK

    def fire_idx(j, slot):
        pltpu.async_copy(dstf_hbm.at[pl.ds(base0 + j * _CHUNK, _CHUNK)],
                         dstv[slot], qsem[slot])

    def wait_idx(j, slot):
        pltpu.make_async_copy(dstf_hbm.at[pl.ds(base0 + j * _CHUNK, _CHUNK)],
                              dstv[slot], qsem[slot]).wait()

    @pl.loop(0, _ZR)
    def _(i):
        for j in range(_H // 16):
            zbuf[i, pl.ds(j * 16, 16)] = jnp.zeros((16,), jnp.float32)

    @pl.loop(0, _CHUNK)
    def _(i):
        for j in range(_H // 16):
            ones_v[i, pl.ds(j * 16, 16)] = jnp.full((16,), 1.0, jnp.float32)

    for k in range(_RPT // _ZR):
        pltpu.sync_copy(zbuf, acc_sh.at[pl.ds(s * _RPT + k * _ZR, _ZR), :])

    for slot in range(4):
        fire_idx(slot, slot)
    plsc.subcore_barrier()

    for b in range(2):
        wait_idx(b, b)
        pltpu.async_copy(ones_v, acc_sh.at[dstv[b]], ssem[b], add=True)

    @pl.loop(2, _CPT - 2, step=4)
    def _(i):
        for u in range(4):
            j = i + u
            slot = (2 + u) % 4
            b = u % 2
            pltpu.make_async_copy(ones_v, acc_sh.at[dstv[(slot + 2) % 4]],
                                  ssem[b]).wait()
            fire_idx(j + 2, (slot + 2) % 4)
            wait_idx(j, slot)
            pltpu.async_copy(ones_v, acc_sh.at[dstv[slot]], ssem[b],
                             add=True)

    for b in range(2):
        j = _CPT - 2 + b
        pltpu.make_async_copy(ones_v, acc_sh.at[dstv[b]], ssem[b]).wait()
        wait_idx(j, j % 4)
        pltpu.async_copy(ones_v, acc_sh.at[dstv[j % 4]], ssem[b], add=True)
    for b in range(2):
        pltpu.make_async_copy(ones_v, acc_sh.at[dstv[2 + b]], ssem[b]).wait()

    plsc.subcore_barrier()
    pltpu.sync_copy(acc_sh.at[pl.ds(s * _RPT, _RPT), :],
                    out_hbm.at[c, pl.ds(s * _RPT, _RPT), :])


@functools.partial(
    pl.kernel,
    out_type=jax.ShapeDtypeStruct((_NC, _NP, _H), jnp.float32),
    mesh=_sc_mesh,
    scratch_types=[
        pltpu.VMEM((_CHUNK,), jnp.int32),
        pltpu.VMEM((_CHUNK,), jnp.int32),
        pltpu.VMEM((_CHUNK,), jnp.int32),
        pltpu.VMEM((_CHUNK,), jnp.int32),
        pltpu.VMEM((_CHUNK,), jnp.int32),
        pltpu.VMEM((_CHUNK,), jnp.int32),
        pltpu.VMEM((_CHUNK,), jnp.int32),
        pltpu.VMEM((_CHUNK,), jnp.int32),
        pltpu.VMEM((_CHUNK, _H), jnp.float32),
        pltpu.VMEM((_CHUNK, _H), jnp.float32),
        pltpu.VMEM((_TAIL,), jnp.int32),
        pltpu.VMEM((_TAIL,), jnp.int32),
        pltpu.VMEM((_TAIL, _H), jnp.float32),
        pltpu.VMEM_SHARED((_NP, _H), jnp.float32),
        pltpu.SemaphoreType.DMA,
        pltpu.SemaphoreType.DMA,
        pltpu.SemaphoreType.DMA,
        pltpu.SemaphoreType.DMA,
        pltpu.SemaphoreType.DMA,
        pltpu.SemaphoreType.DMA,
        pltpu.SemaphoreType.DMA,
        pltpu.SemaphoreType.DMA,
        pltpu.SemaphoreType.DMA,
        pltpu.SemaphoreType.DMA,
        pltpu.SemaphoreType.DMA,
        pltpu.SemaphoreType.DMA,
    ],
)
def _agg_call(hp_hbm, src_hbm, dst_hbm, out_hbm,
              sv0, sv1, sv2, sv3, dv0, dv1, dv2, dv3, r0, r1,
              src_t, dst_t, rows_t, acc_sh,
              p0, p1, p2, p3, q0, q1, q2, q3, g0, g1, t0, t1):
    c = lax.axis_index("c")
    s = lax.axis_index("s")
    wid = c * _NS + s
    srcv = (sv0, sv1, sv2, sv3)
    dstv = (dv0, dv1, dv2, dv3)
    psem = (p0, p1, p2, p3)
    qsem = (q0, q1, q2, q3)
    rows = (r0, r1)
    gsem = (g0, g1)
    ssem = (t0, t1)
    base0 = wid * _EPT

    def fire_idx(j, slot):
        pltpu.async_copy(src_hbm.at[pl.ds(base0 + j * _CHUNK, _CHUNK)],
                         srcv[slot], psem[slot])
        pltpu.async_copy(dst_hbm.at[pl.ds(base0 + j * _CHUNK, _CHUNK)],
                         dstv[slot], qsem[slot])

    def wait_idx(j, slot):
        pltpu.make_async_copy(src_hbm.at[pl.ds(base0 + j * _CHUNK, _CHUNK)],
                              srcv[slot], psem[slot]).wait()
        pltpu.make_async_copy(dst_hbm.at[pl.ds(base0 + j * _CHUNK, _CHUNK)],
                              dstv[slot], qsem[slot]).wait()

    @pl.loop(0, _CHUNK)
    def _(i):
        for j in range(_H // 16):
            r0[i, pl.ds(j * 16, 16)] = jnp.zeros((16,), jnp.float32)

    for k in range(_RPT // _ZR):
        pltpu.sync_copy(r0, acc_sh.at[pl.ds(s * _RPT + k * _ZR, _ZR), :])

    for slot in range(4):
        fire_idx(slot, slot)
    plsc.subcore_barrier()

    for b in range(2):
        wait_idx(b, b)
        pltpu.async_copy(hp_hbm.at[srcv[b]], rows[b], gsem[b]).wait()
        pltpu.async_copy(rows[b], acc_sh.at[dstv[b]], ssem[b], add=True)

    @pl.loop(2, _NFULL, step=4)
    def _(i):
        for u in range(4):
            j = i + u
            slot = (2 + u) % 4
            b = u % 2
            pltpu.make_async_copy(rows[b], acc_sh.at[dstv[(slot + 2) % 4]],
                                  ssem[b]).wait()
            fire_idx(j + 2, (slot + 2) % 4)
            wait_idx(j, slot)
            pltpu.async_copy(hp_hbm.at[srcv[slot]], rows[b], gsem[b]).wait()
            pltpu.async_copy(rows[b], acc_sh.at[dstv[slot]], ssem[b],
                             add=True)

    for b in range(2):
        pltpu.make_async_copy(rows[b], acc_sh.at[dstv[b]], ssem[b]).wait()
    wait_idx(_NFULL, 2)
    wait_idx(_NFULL + 1, 3)

    tbase = base0 + _NFULL * _CHUNK
    pltpu.sync_copy(src_hbm.at[pl.ds(tbase, _TAIL)], src_t)
    pltpu.sync_copy(dst_hbm.at[pl.ds(tbase, _TAIL)], dst_t)
    pltpu.async_copy(hp_hbm.at[src_t], rows_t, gsem[0]).wait()
    pltpu.sync_copy(rows_t, acc_sh.at[dst_t], add=True)

    plsc.subcore_barrier()
    pltpu.sync_copy(acc_sh.at[pl.ds(s * _RPT, _RPT), :],
                    out_hbm.at[c, pl.ds(s * _RPT, _RPT), :])


_PREC = lax.Precision.HIGHEST


def _mm1_body(p0_ref, p1_ref, x_ref, w_ref, oh_ref, od_ref):
    deg = 1.0 + p0_ref[...] + p1_ref[...]
    dinv = lax.rsqrt(deg)
    g = jnp.dot(x_ref[...], w_ref[...],
                preferred_element_type=jnp.float32, precision=_PREC)
    oh_ref[...] = g * dinv
    od_ref[...] = dinv


@jax.jit
def _mm1_call(p0, p1, x, w):
    grid = (_N // _ROWB,)
    return pl.pallas_call(
        _mm1_body,
        grid=grid,
        in_specs=[
            pl.BlockSpec((_ROWB, 1), lambda i: (i, 0)),
            pl.BlockSpec((_ROWB, 1), lambda i: (i, 0)),
            pl.BlockSpec((_ROWB, _H), lambda i: (i, 0)),
            pl.BlockSpec((_H, _H), lambda i: (0, 0)),
        ],
        out_specs=[
            pl.BlockSpec((_ROWB, _H), lambda i: (i, 0)),
            pl.BlockSpec((_ROWB, 1), lambda i: (i, 0)),
        ],
        out_shape=[
            jax.ShapeDtypeStruct((_N, _H), jnp.float32),
            jax.ShapeDtypeStruct((_N, 1), jnp.float32),
        ],
    )(p0, p1, x, w)


def _layer_body(s0_ref, s1_ref, hp_ref, d_ref, b_ref, w_ref, o_ref):
    y = d_ref[...] * (s0_ref[...] + s1_ref[...] + hp_ref[...]) + b_ref[...]
    y = jnp.maximum(y, 0.0)
    o_ref[...] = jnp.dot(y, w_ref[...],
                         preferred_element_type=jnp.float32,
                         precision=_PREC) * d_ref[...]


@jax.jit
def _layer_call(s0, s1, hp, dinv, b, w):
    grid = (_N // _ROWB,)
    return pl.pallas_call(
        _layer_body,
        grid=grid,
        in_specs=[
            pl.BlockSpec((_ROWB, _H), lambda i: (i, 0)),
            pl.BlockSpec((_ROWB, _H), lambda i: (i, 0)),
            pl.BlockSpec((_ROWB, _H), lambda i: (i, 0)),
            pl.BlockSpec((_ROWB, 1), lambda i: (i, 0)),
            pl.BlockSpec((1, _H), lambda i: (0, 0)),
            pl.BlockSpec((_H, _H), lambda i: (0, 0)),
        ],
        out_specs=pl.BlockSpec((_ROWB, _H), lambda i: (i, 0)),
        out_shape=jax.ShapeDtypeStruct((_N, _H), jnp.float32),
    )(s0, s1, hp, dinv, b, w)


def _final_body(s0_ref, s1_ref, hp_ref, d_ref, b_ref, wl_ref, bl_ref, o_ref):
    y = d_ref[...] * (s0_ref[...] + s1_ref[...] + hp_ref[...]) + b_ref[...]
    y = jnp.maximum(y, 0.0)
    o_ref[...] = jnp.dot(y, wl_ref[...],
                         preferred_element_type=jnp.float32,
                         precision=_PREC) + bl_ref[...]


@jax.jit
def _final_call(s0, s1, hp, dinv, b, wl, bl):
    grid = (_N // _ROWB,)
    return pl.pallas_call(
        _final_body,
        grid=grid,
        in_specs=[
            pl.BlockSpec((_ROWB, _H), lambda i: (i, 0)),
            pl.BlockSpec((_ROWB, _H), lambda i: (i, 0)),
            pl.BlockSpec((_ROWB, _H), lambda i: (i, 0)),
            pl.BlockSpec((_ROWB, 1), lambda i: (i, 0)),
            pl.BlockSpec((1, _H), lambda i: (0, 0)),
            pl.BlockSpec((_H, _C), lambda i: (0, 0)),
            pl.BlockSpec((1, _C), lambda i: (0, 0)),
        ],
        out_specs=pl.BlockSpec((_ROWB, _C), lambda i: (i, 0)),
        out_shape=jax.ShapeDtypeStruct((_N, _C), jnp.float32),
    )(s0, s1, hp, dinv, b, wl, bl)


@jax.jit
def kernel(x, edge_index, W1, b1, W2, b2, W3, b3, Wl, bl):
    srcr = edge_index[0]
    dstr = edge_index[1]
    src = jnp.concatenate([srcr, jnp.zeros((2 * _CHUNK,), jnp.int32)])
    dst = jnp.concatenate([dstr, jnp.zeros((2 * _CHUNK,), jnp.int32)])
    pad = _CPT * _CHUNK - _EPT
    dstf = jnp.concatenate([
        jnp.pad(dstr.reshape(_NW, _EPT), ((0, 0), (0, pad)),
                constant_values=_NP - 1).reshape(_NW * _CPT * _CHUNK),
        jnp.zeros((2 * _CHUNK,), jnp.int32)])
    degp = _deg_call(dstf)
    p0 = degp[0, :_N, :1]
    p1 = degp[1, :_N, :1]
    h1, dinv = _mm1_call(p0, p1, x, W1)
    s = _agg_call(h1, src, dst)
    h2 = _layer_call(s[0, :_N], s[1, :_N], h1, dinv, b1.reshape(1, _H), W2)
    s = _agg_call(h2, src, dst)
    h3 = _layer_call(s[0, :_N], s[1, :_N], h2, dinv, b2.reshape(1, _H), W3)
    s = _agg_call(h3, src, dst)
    out = _final_call(s[0, :_N], s[1, :_N], h3, dinv, b3.reshape(1, _H), Wl,
                      bl.reshape(1, _C))
    return out


# speedup vs baseline: 2.8658x; 1.0396x over previous
import functools

import jax
import jax.numpy as jnp
from jax import lax
from jax.experimental import pallas as pl
from jax.experimental.pallas import tpu as pltpu
from jax.experimental.pallas import tpu_sc as plsc

_N = 10000
_E = 320000
_H = 128
_C = 40
_NC = 2
_NS = 16
_NW = _NC * _NS
_CHUNK = 128
_CPT = 80
_NCH = _NW * _CPT
_EPT = _E // _NW
_NFULL = _EPT // _CHUNK
_TAIL = _EPT - _NFULL * _CHUNK
_NP = 10240
_RPT = _NP // _NS
_ZR = 128
_NBUF = 2
_ROWB = 2000

_sc_mesh = plsc.VectorSubcoreMesh(core_axis_name="c", subcore_axis_name="s")


@functools.partial(
    pl.kernel,
    out_type=jax.ShapeDtypeStruct((_NC, _NP, _H), jnp.float32),
    mesh=_sc_mesh,
    scratch_types=[
        pltpu.VMEM((_CHUNK,), jnp.int32),
        pltpu.VMEM((_CHUNK,), jnp.int32),
        pltpu.VMEM((_CHUNK,), jnp.int32),
        pltpu.VMEM((_CHUNK,), jnp.int32),
        pltpu.VMEM((_CHUNK, _H), jnp.float32),
        pltpu.VMEM((_ZR, _H), jnp.float32),
        pltpu.VMEM_SHARED((_NP, _H), jnp.float32),
        pltpu.SemaphoreType.DMA,
        pltpu.SemaphoreType.DMA,
        pltpu.SemaphoreType.DMA,
        pltpu.SemaphoreType.DMA,
        pltpu.SemaphoreType.DMA,
        pltpu.SemaphoreType.DMA,
    ],
)
def _deg_call(dstf_hbm, out_hbm, dv0, dv1, dv2, dv3, ones_v, zbuf, acc_sh,
              q0, q1, q2, q3, t0, t1):
    c = lax.axis_index("c")
    s = lax.axis_index("s")
    wid = c * _NS + s
    dstv = (dv0, dv1, dv2, dv3)
    qsem = (q0, q1, q2, q3)
    ssem = (t0, t1)
    base0 = wid * _CPT * _CHUNK

    def fire_idx(j, slot):
        pltpu.async_copy(dstf_hbm.at[pl.ds(base0 + j * _CHUNK, _CHUNK)],
                         dstv[slot], qsem[slot])

    def wait_idx(j, slot):
        pltpu.make_async_copy(dstf_hbm.at[pl.ds(base0 + j * _CHUNK, _CHUNK)],
                              dstv[slot], qsem[slot]).wait()

    @pl.loop(0, _ZR)
    def _(i):
        for j in range(_H // 16):
            zbuf[i, pl.ds(j * 16, 16)] = jnp.zeros((16,), jnp.float32)

    @pl.loop(0, _CHUNK)
    def _(i):
        for j in range(_H // 16):
            ones_v[i, pl.ds(j * 16, 16)] = jnp.full((16,), 1.0, jnp.float32)

    for k in range(_RPT // _ZR):
        pltpu.sync_copy(zbuf, acc_sh.at[pl.ds(s * _RPT + k * _ZR, _ZR), :])

    for slot in range(4):
        fire_idx(slot, slot)
    plsc.subcore_barrier()

    for b in range(2):
        wait_idx(b, b)
        pltpu.async_copy(ones_v, acc_sh.at[dstv[b]], ssem[b], add=True)

    @pl.loop(2, _CPT - 2, step=4)
    def _(i):
        for u in range(4):
            j = i + u
            slot = (2 + u) % 4
            b = u % 2
            pltpu.make_async_copy(ones_v, acc_sh.at[dstv[(slot + 2) % 4]],
                                  ssem[b]).wait()
            fire_idx(j + 2, (slot + 2) % 4)
            wait_idx(j, slot)
            pltpu.async_copy(ones_v, acc_sh.at[dstv[slot]], ssem[b],
                             add=True)

    for b in range(2):
        j = _CPT - 2 + b
        pltpu.make_async_copy(ones_v, acc_sh.at[dstv[b]], ssem[b]).wait()
        wait_idx(j, j % 4)
        pltpu.async_copy(ones_v, acc_sh.at[dstv[j % 4]], ssem[b], add=True)
    for b in range(2):
        pltpu.make_async_copy(ones_v, acc_sh.at[dstv[2 + b]], ssem[b]).wait()

    plsc.subcore_barrier()
    pltpu.sync_copy(acc_sh.at[pl.ds(s * _RPT, _RPT), :],
                    out_hbm.at[c, pl.ds(s * _RPT, _RPT), :])


@functools.partial(
    pl.kernel,
    out_type=jax.ShapeDtypeStruct((_NC, _NP, _H), jnp.float32),
    mesh=_sc_mesh,
    scratch_types=[
        pltpu.VMEM((_CHUNK,), jnp.int32),
        pltpu.VMEM((_CHUNK,), jnp.int32),
        pltpu.VMEM((_CHUNK,), jnp.int32),
        pltpu.VMEM((_CHUNK,), jnp.int32),
        pltpu.VMEM((_CHUNK,), jnp.int32),
        pltpu.VMEM((_CHUNK,), jnp.int32),
        pltpu.VMEM((_CHUNK,), jnp.int32),
        pltpu.VMEM((_CHUNK,), jnp.int32),
        pltpu.VMEM((_CHUNK, _H), jnp.float32),
        pltpu.VMEM((_CHUNK, _H), jnp.float32),
        pltpu.VMEM((_TAIL,), jnp.int32),
        pltpu.VMEM((_TAIL,), jnp.int32),
        pltpu.VMEM((_TAIL, _H), jnp.float32),
        pltpu.VMEM_SHARED((_NP, _H), jnp.float32),
        pltpu.SemaphoreType.DMA,
        pltpu.SemaphoreType.DMA,
        pltpu.SemaphoreType.DMA,
        pltpu.SemaphoreType.DMA,
        pltpu.SemaphoreType.DMA,
        pltpu.SemaphoreType.DMA,
        pltpu.SemaphoreType.DMA,
        pltpu.SemaphoreType.DMA,
        pltpu.SemaphoreType.DMA,
        pltpu.SemaphoreType.DMA,
        pltpu.SemaphoreType.DMA,
        pltpu.SemaphoreType.DMA,
    ],
)
def _agg_call(hp_hbm, src_hbm, dst_hbm, out_hbm,
              sv0, sv1, sv2, sv3, dv0, dv1, dv2, dv3, r0, r1,
              src_t, dst_t, rows_t, acc_sh,
              p0, p1, p2, p3, q0, q1, q2, q3, g0, g1, t0, t1):
    c = lax.axis_index("c")
    s = lax.axis_index("s")
    wid = c * _NS + s
    srcv = (sv0, sv1, sv2, sv3)
    dstv = (dv0, dv1, dv2, dv3)
    psem = (p0, p1, p2, p3)
    qsem = (q0, q1, q2, q3)
    rows = (r0, r1)
    gsem = (g0, g1)
    ssem = (t0, t1)
    base0 = wid * _EPT

    def fire_idx(j, slot):
        pltpu.async_copy(src_hbm.at[pl.ds(base0 + j * _CHUNK, _CHUNK)],
                         srcv[slot], psem[slot])
        pltpu.async_copy(dst_hbm.at[pl.ds(base0 + j * _CHUNK, _CHUNK)],
                         dstv[slot], qsem[slot])

    def wait_idx(j, slot):
        pltpu.make_async_copy(src_hbm.at[pl.ds(base0 + j * _CHUNK, _CHUNK)],
                              srcv[slot], psem[slot]).wait()
        pltpu.make_async_copy(dst_hbm.at[pl.ds(base0 + j * _CHUNK, _CHUNK)],
                              dstv[slot], qsem[slot]).wait()

    @pl.loop(0, _CHUNK)
    def _(i):
        for j in range(_H // 16):
            r0[i, pl.ds(j * 16, 16)] = jnp.zeros((16,), jnp.float32)

    for k in range(_RPT // _ZR):
        pltpu.sync_copy(r0, acc_sh.at[pl.ds(s * _RPT + k * _ZR, _ZR), :])

    for slot in range(4):
        fire_idx(slot, slot)
    plsc.subcore_barrier()

    for b in range(2):
        wait_idx(b, b)
        pltpu.async_copy(hp_hbm.at[srcv[b]], rows[b], gsem[b]).wait()
        pltpu.async_copy(rows[b], acc_sh.at[dstv[b]], ssem[b], add=True)

    @pl.loop(2, _NFULL, step=4)
    def _(i):
        for u in range(4):
            j = i + u
            slot = (2 + u) % 4
            b = u % 2
            pltpu.make_async_copy(rows[b], acc_sh.at[dstv[(slot + 2) % 4]],
                                  ssem[b]).wait()
            fire_idx(j + 2, (slot + 2) % 4)
            wait_idx(j, slot)
            pltpu.async_copy(hp_hbm.at[srcv[slot]], rows[b], gsem[b]).wait()
            pltpu.async_copy(rows[b], acc_sh.at[dstv[slot]], ssem[b],
                             add=True)

    for b in range(2):
        pltpu.make_async_copy(rows[b], acc_sh.at[dstv[b]], ssem[b]).wait()
    wait_idx(_NFULL, 2)
    wait_idx(_NFULL + 1, 3)

    tbase = base0 + _NFULL * _CHUNK
    pltpu.sync_copy(src_hbm.at[pl.ds(tbase, _TAIL)], src_t)
    pltpu.sync_copy(dst_hbm.at[pl.ds(tbase, _TAIL)], dst_t)
    pltpu.async_copy(hp_hbm.at[src_t], rows_t, gsem[0]).wait()
    pltpu.sync_copy(rows_t, acc_sh.at[dst_t], add=True)

    plsc.subcore_barrier()
    pltpu.sync_copy(acc_sh.at[pl.ds(s * _RPT, _RPT), :],
                    out_hbm.at[c, pl.ds(s * _RPT, _RPT), :])


_PREC = lax.Precision.HIGHEST


def _mm1_body(p0_ref, p1_ref, x_ref, w_ref, oh_ref, od_ref):
    deg = 1.0 + p0_ref[...] + p1_ref[...]
    dinv = lax.rsqrt(deg)
    g = jnp.dot(x_ref[...], w_ref[...],
                preferred_element_type=jnp.float32, precision=_PREC)
    oh_ref[...] = g * dinv
    od_ref[...] = dinv


@jax.jit
def _mm1_call(p0, p1, x, w):
    grid = (_N // _ROWB,)
    return pl.pallas_call(
        _mm1_body,
        grid=grid,
        in_specs=[
            pl.BlockSpec((_ROWB, 1), lambda i: (i, 0)),
            pl.BlockSpec((_ROWB, 1), lambda i: (i, 0)),
            pl.BlockSpec((_ROWB, _H), lambda i: (i, 0)),
            pl.BlockSpec((_H, _H), lambda i: (0, 0)),
        ],
        out_specs=[
            pl.BlockSpec((_ROWB, _H), lambda i: (i, 0)),
            pl.BlockSpec((_ROWB, 1), lambda i: (i, 0)),
        ],
        out_shape=[
            jax.ShapeDtypeStruct((_N, _H), jnp.float32),
            jax.ShapeDtypeStruct((_N, 1), jnp.float32),
        ],
    )(p0, p1, x, w)


def _layer_body(s_ref, hp_ref, d_ref, b_ref, w_ref, o_ref):
    ssum = s_ref[0] + s_ref[1]
    y = d_ref[...] * (ssum + hp_ref[...]) + b_ref[...]
    y = jnp.maximum(y, 0.0)
    o_ref[...] = jnp.dot(y, w_ref[...],
                         preferred_element_type=jnp.float32,
                         precision=_PREC) * d_ref[...]


@jax.jit
def _layer_call(s, hp, dinv, b, w):
    grid = (_N // _ROWB,)
    return pl.pallas_call(
        _layer_body,
        grid=grid,
        in_specs=[
            pl.BlockSpec((_NC, _ROWB, _H), lambda i: (0, i, 0)),
            pl.BlockSpec((_ROWB, _H), lambda i: (i, 0)),
            pl.BlockSpec((_ROWB, 1), lambda i: (i, 0)),
            pl.BlockSpec((1, _H), lambda i: (0, 0)),
            pl.BlockSpec((_H, _H), lambda i: (0, 0)),
        ],
        out_specs=pl.BlockSpec((_ROWB, _H), lambda i: (i, 0)),
        out_shape=jax.ShapeDtypeStruct((_N, _H), jnp.float32),
    )(s, hp, dinv, b, w)


def _final_body(s_ref, hp_ref, d_ref, b_ref, wl_ref, bl_ref, o_ref):
    ssum = s_ref[0] + s_ref[1]
    y = d_ref[...] * (ssum + hp_ref[...]) + b_ref[...]
    y = jnp.maximum(y, 0.0)
    o_ref[...] = jnp.dot(y, wl_ref[...],
                         preferred_element_type=jnp.float32,
                         precision=_PREC) + bl_ref[...]


@jax.jit
def _final_call(s, hp, dinv, b, wl, bl):
    grid = (_N // _ROWB,)
    return pl.pallas_call(
        _final_body,
        grid=grid,
        in_specs=[
            pl.BlockSpec((_NC, _ROWB, _H), lambda i: (0, i, 0)),
            pl.BlockSpec((_ROWB, _H), lambda i: (i, 0)),
            pl.BlockSpec((_ROWB, 1), lambda i: (i, 0)),
            pl.BlockSpec((1, _H), lambda i: (0, 0)),
            pl.BlockSpec((_H, _C), lambda i: (0, 0)),
            pl.BlockSpec((1, _C), lambda i: (0, 0)),
        ],
        out_specs=pl.BlockSpec((_ROWB, _C), lambda i: (i, 0)),
        out_shape=jax.ShapeDtypeStruct((_N, _C), jnp.float32),
    )(s, hp, dinv, b, wl, bl)


@jax.jit
def kernel(x, edge_index, W1, b1, W2, b2, W3, b3, Wl, bl):
    srcr = edge_index[0]
    dstr = edge_index[1]
    src = jnp.concatenate([srcr, jnp.zeros((2 * _CHUNK,), jnp.int32)])
    dst = jnp.concatenate([dstr, jnp.zeros((2 * _CHUNK,), jnp.int32)])
    pad = _CPT * _CHUNK - _EPT
    dstf = jnp.concatenate([
        jnp.pad(dstr.reshape(_NW, _EPT), ((0, 0), (0, pad)),
                constant_values=_NP - 1).reshape(_NW * _CPT * _CHUNK),
        jnp.zeros((2 * _CHUNK,), jnp.int32)])
    degp = _deg_call(dstf)
    p0 = degp[0, :_N, :1]
    p1 = degp[1, :_N, :1]
    h1, dinv = _mm1_call(p0, p1, x, W1)
    s = _agg_call(h1, src, dst)
    h2 = _layer_call(s, h1, dinv, b1.reshape(1, _H), W2)
    s = _agg_call(h2, src, dst)
    h3 = _layer_call(s, h2, dinv, b2.reshape(1, _H), W3)
    s = _agg_call(h3, src, dst)
    out = _final_call(s, h3, dinv, b3.reshape(1, _H), Wl,
                      bl.reshape(1, _C))
    return out


# speedup vs baseline: 2.9041x; 1.0134x over previous
import functools

import jax
import jax.numpy as jnp
from jax import lax
from jax.experimental import pallas as pl
from jax.experimental.pallas import tpu as pltpu
from jax.experimental.pallas import tpu_sc as plsc

_N = 10000
_E = 320000
_H = 128
_C = 40
_NC = 2
_NS = 16
_NW = _NC * _NS
_CHUNK = 128
_CPT = 80
_NCH = _NW * _CPT
_EPT = _E // _NW
_NFULL = _EPT // _CHUNK
_TAIL = _EPT - _NFULL * _CHUNK
_NP = 10240
_RPT = _NP // _NS
_ZR = 128
_NBUF = 2
_ROWB = 2000

_sc_mesh = plsc.VectorSubcoreMesh(core_axis_name="c", subcore_axis_name="s")


@functools.partial(
    pl.kernel,
    out_type=jax.ShapeDtypeStruct((_NC, _NP, _H), jnp.float32),
    mesh=_sc_mesh,
    scratch_types=[
        pltpu.VMEM((_CHUNK,), jnp.int32),
        pltpu.VMEM((_CHUNK,), jnp.int32),
        pltpu.VMEM((_CHUNK,), jnp.int32),
        pltpu.VMEM((_CHUNK,), jnp.int32),
        pltpu.VMEM((_CHUNK, _H), jnp.float32),
        pltpu.VMEM((_ZR, _H), jnp.float32),
        pltpu.VMEM_SHARED((_NP, _H), jnp.float32),
        pltpu.SemaphoreType.DMA,
        pltpu.SemaphoreType.DMA,
        pltpu.SemaphoreType.DMA,
        pltpu.SemaphoreType.DMA,
        pltpu.SemaphoreType.DMA,
        pltpu.SemaphoreType.DMA,
    ],
)
def _deg_call(dstf_hbm, out_hbm, dv0, dv1, dv2, dv3, ones_v, zbuf, acc_sh,
              q0, q1, q2, q3, t0, t1):
    c = lax.axis_index("c")
    s = lax.axis_index("s")
    wid = c * _NS + s
    dstv = (dv0, dv1, dv2, dv3)
    qsem = (q0, q1, q2, q3)
    ssem = (t0, t1)
    base0 = wid * _CPT * _CHUNK

    def fire_idx(j, slot):
        pltpu.async_copy(dstf_hbm.at[pl.ds(base0 + j * _CHUNK, _CHUNK)],
                         dstv[slot], qsem[slot])

    def wait_idx(j, slot):
        pltpu.make_async_copy(dstf_hbm.at[pl.ds(base0 + j * _CHUNK, _CHUNK)],
                              dstv[slot], qsem[slot]).wait()

    @pl.loop(0, _ZR)
    def _(i):
        for j in range(_H // 16):
            zbuf[i, pl.ds(j * 16, 16)] = jnp.zeros((16,), jnp.float32)

    @pl.loop(0, _CHUNK)
    def _(i):
        for j in range(_H // 16):
            ones_v[i, pl.ds(j * 16, 16)] = jnp.full((16,), 1.0, jnp.float32)

    for k in range(_RPT // _ZR):
        pltpu.sync_copy(zbuf, acc_sh.at[pl.ds(s * _RPT + k * _ZR, _ZR), :])

    for slot in range(4):
        fire_idx(slot, slot)
    plsc.subcore_barrier()

    for b in range(2):
        wait_idx(b, b)
        pltpu.async_copy(ones_v, acc_sh.at[dstv[b]], ssem[b], add=True)

    @pl.loop(2, _CPT - 2, step=4)
    def _(i):
        for u in range(4):
            j = i + u
            slot = (2 + u) % 4
            b = u % 2
            pltpu.make_async_copy(ones_v, acc_sh.at[dstv[(slot + 2) % 4]],
                                  ssem[b]).wait()
            fire_idx(j + 2, (slot + 2) % 4)
            wait_idx(j, slot)
            pltpu.async_copy(ones_v, acc_sh.at[dstv[slot]], ssem[b],
                             add=True)

    for b in range(2):
        j = _CPT - 2 + b
        pltpu.make_async_copy(ones_v, acc_sh.at[dstv[b]], ssem[b]).wait()
        wait_idx(j, j % 4)
        pltpu.async_copy(ones_v, acc_sh.at[dstv[j % 4]], ssem[b], add=True)
    for b in range(2):
        pltpu.make_async_copy(ones_v, acc_sh.at[dstv[2 + b]], ssem[b]).wait()

    plsc.subcore_barrier()
    pltpu.sync_copy(acc_sh.at[pl.ds(s * _RPT, _RPT), :],
                    out_hbm.at[c, pl.ds(s * _RPT, _RPT), :])


@functools.partial(
    pl.kernel,
    out_type=jax.ShapeDtypeStruct((_NC, _NP, _H), jnp.float32),
    mesh=_sc_mesh,
    scratch_types=[
        pltpu.VMEM((_CHUNK,), jnp.int32),
        pltpu.VMEM((_CHUNK,), jnp.int32),
        pltpu.VMEM((_CHUNK,), jnp.int32),
        pltpu.VMEM((_CHUNK,), jnp.int32),
        pltpu.VMEM((_CHUNK,), jnp.int32),
        pltpu.VMEM((_CHUNK,), jnp.int32),
        pltpu.VMEM((_CHUNK,), jnp.int32),
        pltpu.VMEM((_CHUNK,), jnp.int32),
        pltpu.VMEM((_CHUNK, _H), jnp.float32),
        pltpu.VMEM((_CHUNK, _H), jnp.float32),
        pltpu.VMEM((_TAIL,), jnp.int32),
        pltpu.VMEM((_TAIL,), jnp.int32),
        pltpu.VMEM((_TAIL, _H), jnp.float32),
        pltpu.VMEM_SHARED((_NP, _H), jnp.float32),
        pltpu.SemaphoreType.DMA,
        pltpu.SemaphoreType.DMA,
        pltpu.SemaphoreType.DMA,
        pltpu.SemaphoreType.DMA,
        pltpu.SemaphoreType.DMA,
        pltpu.SemaphoreType.DMA,
        pltpu.SemaphoreType.DMA,
        pltpu.SemaphoreType.DMA,
        pltpu.SemaphoreType.DMA,
        pltpu.SemaphoreType.DMA,
        pltpu.SemaphoreType.DMA,
        pltpu.SemaphoreType.DMA,
    ],
)
def _agg_call(hp_hbm, src_hbm, dst_hbm, out_hbm,
              sv0, sv1, sv2, sv3, dv0, dv1, dv2, dv3, r0, r1,
              src_t, dst_t, rows_t, acc_sh,
              p0, p1, p2, p3, q0, q1, q2, q3, g0, g1, t0, t1):
    c = lax.axis_index("c")
    s = lax.axis_index("s")
    wid = c * _NS + s
    srcv = (sv0, sv1, sv2, sv3)
    dstv = (dv0, dv1, dv2, dv3)
    psem = (p0, p1, p2, p3)
    qsem = (q0, q1, q2, q3)
    rows = (r0, r1)
    gsem = (g0, g1)
    ssem = (t0, t1)
    base0 = wid * _EPT

    def fire_idx(j, slot):
        pltpu.async_copy(src_hbm.at[pl.ds(base0 + j * _CHUNK, _CHUNK)],
                         srcv[slot], psem[slot])
        pltpu.async_copy(dst_hbm.at[pl.ds(base0 + j * _CHUNK, _CHUNK)],
                         dstv[slot], qsem[slot])

    def wait_idx(j, slot):
        pltpu.make_async_copy(src_hbm.at[pl.ds(base0 + j * _CHUNK, _CHUNK)],
                              srcv[slot], psem[slot]).wait()
        pltpu.make_async_copy(dst_hbm.at[pl.ds(base0 + j * _CHUNK, _CHUNK)],
                              dstv[slot], qsem[slot]).wait()

    @pl.loop(0, _CHUNK)
    def _(i):
        for j in range(_H // 16):
            r0[i, pl.ds(j * 16, 16)] = jnp.zeros((16,), jnp.float32)

    for k in range(_RPT // _ZR):
        pltpu.sync_copy(r0, acc_sh.at[pl.ds(s * _RPT + k * _ZR, _ZR), :])

    for slot in range(4):
        fire_idx(slot, slot)
    plsc.subcore_barrier()

    for b in range(2):
        wait_idx(b, b)
        pltpu.async_copy(hp_hbm.at[srcv[b]], rows[b], gsem[b]).wait()
        pltpu.async_copy(rows[b], acc_sh.at[dstv[b]], ssem[b], add=True)

    @pl.loop(2, _NFULL, step=4)
    def _(i):
        for u in range(4):
            j = i + u
            slot = (2 + u) % 4
            b = u % 2
            pltpu.make_async_copy(rows[b], acc_sh.at[dstv[(slot + 2) % 4]],
                                  ssem[b]).wait()
            fire_idx(j + 2, (slot + 2) % 4)
            wait_idx(j, slot)
            pltpu.async_copy(hp_hbm.at[srcv[slot]], rows[b], gsem[b]).wait()
            pltpu.async_copy(rows[b], acc_sh.at[dstv[slot]], ssem[b],
                             add=True)

    for b in range(2):
        pltpu.make_async_copy(rows[b], acc_sh.at[dstv[b]], ssem[b]).wait()
    wait_idx(_NFULL, 2)
    wait_idx(_NFULL + 1, 3)

    tbase = base0 + _NFULL * _CHUNK
    pltpu.sync_copy(src_hbm.at[pl.ds(tbase, _TAIL)], src_t)
    pltpu.sync_copy(dst_hbm.at[pl.ds(tbase, _TAIL)], dst_t)
    pltpu.async_copy(hp_hbm.at[src_t], rows_t, gsem[0]).wait()
    pltpu.sync_copy(rows_t, acc_sh.at[dst_t], add=True)

    plsc.subcore_barrier()
    pltpu.sync_copy(acc_sh.at[pl.ds(s * _RPT, _RPT), :],
                    out_hbm.at[c, pl.ds(s * _RPT, _RPT), :])


_PREC = lax.Precision.HIGHEST


def _g_body(x_ref, w_ref, o_ref):
    o_ref[...] = jnp.dot(x_ref[...], w_ref[...],
                         preferred_element_type=jnp.float32, precision=_PREC)


@jax.jit
def _g_call(x, w):
    grid = (_N // _ROWB,)
    return pl.pallas_call(
        _g_body,
        grid=grid,
        in_specs=[
            pl.BlockSpec((_ROWB, _H), lambda i: (i, 0)),
            pl.BlockSpec((_H, _H), lambda i: (0, 0)),
        ],
        out_specs=pl.BlockSpec((_ROWB, _H), lambda i: (i, 0)),
        out_shape=jax.ShapeDtypeStruct((_N, _H), jnp.float32),
    )(x, w)


def _scale_body(p_ref, g_ref, oh_ref, od_ref):
    deg = 1.0 + p_ref[0, :, :1] + p_ref[1, :, :1]
    dinv = lax.rsqrt(deg)
    oh_ref[...] = g_ref[...] * dinv
    od_ref[...] = dinv


@jax.jit
def _scale_call(degp, g):
    grid = (_N // _ROWB,)
    return pl.pallas_call(
        _scale_body,
        grid=grid,
        in_specs=[
            pl.BlockSpec((_NC, _ROWB, _H), lambda i: (0, i, 0)),
            pl.BlockSpec((_ROWB, _H), lambda i: (i, 0)),
        ],
        out_specs=[
            pl.BlockSpec((_ROWB, _H), lambda i: (i, 0)),
            pl.BlockSpec((_ROWB, 1), lambda i: (i, 0)),
        ],
        out_shape=[
            jax.ShapeDtypeStruct((_N, _H), jnp.float32),
            jax.ShapeDtypeStruct((_N, 1), jnp.float32),
        ],
    )(degp, g)


def _layer_body(s_ref, hp_ref, d_ref, b_ref, w_ref, o_ref):
    ssum = s_ref[0] + s_ref[1]
    y = d_ref[...] * (ssum + hp_ref[...]) + b_ref[...]
    y = jnp.maximum(y, 0.0)
    o_ref[...] = jnp.dot(y, w_ref[...],
                         preferred_element_type=jnp.float32,
                         precision=_PREC) * d_ref[...]


@jax.jit
def _layer_call(s, hp, dinv, b, w):
    grid = (_N // _ROWB,)
    return pl.pallas_call(
        _layer_body,
        grid=grid,
        in_specs=[
            pl.BlockSpec((_NC, _ROWB, _H), lambda i: (0, i, 0)),
            pl.BlockSpec((_ROWB, _H), lambda i: (i, 0)),
            pl.BlockSpec((_ROWB, 1), lambda i: (i, 0)),
            pl.BlockSpec((1, _H), lambda i: (0, 0)),
            pl.BlockSpec((_H, _H), lambda i: (0, 0)),
        ],
        out_specs=pl.BlockSpec((_ROWB, _H), lambda i: (i, 0)),
        out_shape=jax.ShapeDtypeStruct((_N, _H), jnp.float32),
    )(s, hp, dinv, b, w)


def _final_body(s_ref, hp_ref, d_ref, b_ref, wl_ref, bl_ref, o_ref):
    ssum = s_ref[0] + s_ref[1]
    y = d_ref[...] * (ssum + hp_ref[...]) + b_ref[...]
    y = jnp.maximum(y, 0.0)
    o_ref[...] = jnp.dot(y, wl_ref[...],
                         preferred_element_type=jnp.float32,
                         precision=_PREC) + bl_ref[...]


@jax.jit
def _final_call(s, hp, dinv, b, wl, bl):
    grid = (_N // _ROWB,)
    return pl.pallas_call(
        _final_body,
        grid=grid,
        in_specs=[
            pl.BlockSpec((_NC, _ROWB, _H), lambda i: (0, i, 0)),
            pl.BlockSpec((_ROWB, _H), lambda i: (i, 0)),
            pl.BlockSpec((_ROWB, 1), lambda i: (i, 0)),
            pl.BlockSpec((1, _H), lambda i: (0, 0)),
            pl.BlockSpec((_H, _C), lambda i: (0, 0)),
            pl.BlockSpec((1, _C), lambda i: (0, 0)),
        ],
        out_specs=pl.BlockSpec((_ROWB, _C), lambda i: (i, 0)),
        out_shape=jax.ShapeDtypeStruct((_N, _C), jnp.float32),
    )(s, hp, dinv, b, wl, bl)


@jax.jit
def kernel(x, edge_index, W1, b1, W2, b2, W3, b3, Wl, bl):
    srcr = edge_index[0]
    dstr = edge_index[1]
    src = jnp.concatenate([srcr, jnp.zeros((2 * _CHUNK,), jnp.int32)])
    dst = jnp.concatenate([dstr, jnp.zeros((2 * _CHUNK,), jnp.int32)])
    pad = _CPT * _CHUNK - _EPT
    dstf = jnp.concatenate([
        jnp.pad(dstr.reshape(_NW, _EPT), ((0, 0), (0, pad)),
                constant_values=_NP - 1).reshape(_NW * _CPT * _CHUNK),
        jnp.zeros((2 * _CHUNK,), jnp.int32)])
    g1 = _g_call(x, W1)
    degp = _deg_call(dstf)
    h1, dinv = _scale_call(degp, g1)
    s = _agg_call(h1, src, dst)
    h2 = _layer_call(s, h1, dinv, b1.reshape(1, _H), W2)
    s = _agg_call(h2, src, dst)
    h3 = _layer_call(s, h2, dinv, b2.reshape(1, _H), W3)
    s = _agg_call(h3, src, dst)
    out = _final_call(s, h3, dinv, b3.reshape(1, _H), Wl,
                      bl.reshape(1, _C))
    return out
